# Initial kernel scaffold; baseline (speedup 1.0000x reference)
import jax
import jax.numpy as jnp
from jax.experimental import pallas as pl


def kernel(x_lig, edge_index_lig, edge_attr_lig, x_prot, edge_index_prot, edge_attr_prot, edge_index_inter, edge_attr_inter, W_init, b_init, W_msg_h, W_msg_e, b_msg, W_self, W_inter_m, W_inter_e, b_inter, W_fc1, b_fc1, W_fc2, b_fc2):
    raise NotImplementedError("write your pallas kernel here")


# speedup vs baseline: 2.1390x; 2.1390x over previous
import functools

import jax
import jax.numpy as jnp
from jax import lax
from jax.experimental import pallas as pl
from jax.experimental.pallas import tpu as pltpu
from jax.experimental.pallas import tpu_sc as plsc

N = 10000
NG = 2 * N
E = 160000
D = 128
LANES = 16
NSUB = 16
VPR = D // LANES

CH = 128
EPT = E // NSUB
NCH = EPT // CH
TAIL = EPT - NCH * CH
RPT = 624

EPT_I = E // (2 * NSUB)
NCH_I = EPT_I // CH
TAIL_I = EPT_I - NCH_I * CH


@functools.cache
def _mesh():
    return plsc.VectorSubcoreMesh(core_axis_name="c", subcore_axis_name="s",
                                  num_cores=2, num_subcores=NSUB)


def _sc_layer_body(hw_hbm, eaw_hbm, b_hbm, src_hbm, dst_hbm, agg_hbm,
                   idx_s, idx_d, idx_st, idx_dt, rows, eaw_b, bias_b, acc,
                   sem1, sem2, sem3):
    c = lax.axis_index("c")
    s = lax.axis_index("s")
    zero = jnp.zeros((LANES,), jnp.float32)

    pltpu.sync_copy(b_hbm, bias_b)
    bias = [bias_b[0, pl.ds(k * LANES, LANES)] for k in range(VPR)]

    def zbody(r, carry):
        for k in range(VPR):
            rows[r, pl.ds(k * LANES, LANES)] = zero
        return carry

    lax.fori_loop(0, CH, zbody, 0)
    base_r = s * RPT

    @pl.when(s < NSUB - 1)
    def _():
        for j in range(4):
            pltpu.sync_copy(rows, acc.at[pl.ds(base_r + j * CH, CH)])
        pltpu.sync_copy(rows.at[pl.ds(0, 112)],
                        acc.at[pl.ds(base_r + 4 * CH, 112)])

    @pl.when(s == NSUB - 1)
    def _():
        for j in range(5):
            pltpu.sync_copy(rows, acc.at[pl.ds(base_r + j * CH, CH)])

    plsc.subcore_barrier()

    def relu_add(nrows):
        def body(r, carry):
            for k in range(VPR):
                sl = pl.ds(k * LANES, LANES)
                rows[r, sl] = jnp.maximum((rows[r, sl] + eaw_b[r, sl])
                                          + bias[k], zero)
            return carry
        lax.fori_loop(0, nrows, body, 0)

    tile_base = c * E + s * EPT

    def chunk(i, carry):
        base = tile_base + i * CH
        cp_s = pltpu.async_copy(src_hbm.at[pl.ds(base, CH)], idx_s, sem1)
        cp_d = pltpu.async_copy(dst_hbm.at[pl.ds(base, CH)], idx_d, sem2)
        cp_e = pltpu.async_copy(eaw_hbm.at[pl.ds(base, CH)], eaw_b, sem3)
        cp_s.wait()
        pltpu.async_copy(hw_hbm.at[idx_s], rows, sem1).wait()
        cp_e.wait()
        relu_add(CH)
        cp_d.wait()
        pltpu.sync_copy(rows, acc.at[idx_d], add=True)
        return carry

    lax.fori_loop(0, NCH, chunk, 0)

    base = tile_base + NCH * CH
    cp_s = pltpu.async_copy(src_hbm.at[pl.ds(base, TAIL)], idx_st, sem1)
    cp_d = pltpu.async_copy(dst_hbm.at[pl.ds(base, TAIL)], idx_dt, sem2)
    cp_e = pltpu.async_copy(eaw_hbm.at[pl.ds(base, TAIL)],
                            eaw_b.at[pl.ds(0, TAIL)], sem3)
    cp_s.wait()
    pltpu.async_copy(hw_hbm.at[idx_st], rows.at[pl.ds(0, TAIL)], sem1).wait()
    cp_e.wait()
    relu_add(TAIL)
    cp_d.wait()
    pltpu.sync_copy(rows.at[pl.ds(0, TAIL)], acc.at[idx_dt], add=True)

    plsc.subcore_barrier()
    out_base = c * N + base_r

    @pl.when(s < NSUB - 1)
    def _():
        pltpu.sync_copy(acc.at[pl.ds(base_r, RPT)],
                        agg_hbm.at[pl.ds(out_base, RPT)])

    @pl.when(s == NSUB - 1)
    def _():
        pltpu.sync_copy(acc.at[pl.ds(base_r, 640)],
                        agg_hbm.at[pl.ds(out_base, 640)])


@functools.cache
def _sc_layer():
    return pl.kernel(
        _sc_layer_body,
        out_type=[jax.ShapeDtypeStruct((NG, D), jnp.float32)],
        mesh=_mesh(),
        scratch_types=[
            pltpu.VMEM((CH,), jnp.int32),
            pltpu.VMEM((CH,), jnp.int32),
            pltpu.VMEM((TAIL,), jnp.int32),
            pltpu.VMEM((TAIL,), jnp.int32),
            pltpu.VMEM((CH, D), jnp.float32),
            pltpu.VMEM((CH, D), jnp.float32),
            pltpu.VMEM((1, D), jnp.float32),
            pltpu.VMEM_SHARED((N, D), jnp.float32),
            pltpu.SemaphoreType.DMA,
            pltpu.SemaphoreType.DMA,
            pltpu.SemaphoreType.DMA,
        ],
    )


def _sc_mgather_body(h1_hbm, h2_hbm, h3_hbm, src_hbm, dst_hbm, m_hbm,
                     idx_s, idx_d, idx_st, idx_dt, g1, g2, mb,
                     sem1, sem2, sem3):
    c = lax.axis_index("c")
    s = lax.axis_index("s")
    wid = c * NSUB + s
    tile_base = wid * EPT_I
    tabs = (h1_hbm, h2_hbm, h3_hbm)

    def gather_add(nrows, isrc, idst):
        for t, tab in enumerate(tabs):
            cp1 = pltpu.async_copy(tab.at[isrc], g1.at[pl.ds(0, nrows)], sem1)
            cp2 = pltpu.async_copy(tab.at[idst], g2.at[pl.ds(0, nrows)], sem2)
            cp1.wait()
            cp2.wait()

            def rbody(r, carry):
                for k in range(VPR):
                    sl = pl.ds(k * LANES, LANES)
                    mb[r, pl.ds(t * D + k * LANES, LANES)] = (
                        g1[r, sl] + g2[r, sl])
                return carry
            lax.fori_loop(0, nrows, rbody, 0)

    def chunk(i, carry):
        base = tile_base + i * CH
        cp_s = pltpu.async_copy(src_hbm.at[pl.ds(base, CH)], idx_s, sem1)
        cp_d = pltpu.async_copy(dst_hbm.at[pl.ds(base, CH)], idx_d, sem2)
        cp_s.wait()
        cp_d.wait()
        gather_add(CH, idx_s, idx_d)
        pltpu.sync_copy(mb, m_hbm.at[pl.ds(base, CH)])
        return carry

    lax.fori_loop(0, NCH_I, chunk, 0)

    base = tile_base + NCH_I * CH
    cp_s = pltpu.async_copy(src_hbm.at[pl.ds(base, TAIL_I)], idx_st, sem1)
    cp_d = pltpu.async_copy(dst_hbm.at[pl.ds(base, TAIL_I)], idx_dt, sem2)
    cp_s.wait()
    cp_d.wait()
    gather_add(TAIL_I, idx_st, idx_dt)
    pltpu.sync_copy(mb.at[pl.ds(0, TAIL_I)], m_hbm.at[pl.ds(base, TAIL_I)])


@functools.cache
def _sc_mgather():
    return pl.kernel(
        _sc_mgather_body,
        out_type=[jax.ShapeDtypeStruct((E, 3 * D), jnp.float32)],
        mesh=_mesh(),
        scratch_types=[
            pltpu.VMEM((CH,), jnp.int32),
            pltpu.VMEM((CH,), jnp.int32),
            pltpu.VMEM((TAIL_I,), jnp.int32),
            pltpu.VMEM((TAIL_I,), jnp.int32),
            pltpu.VMEM((CH, D), jnp.float32),
            pltpu.VMEM((CH, D), jnp.float32),
            pltpu.VMEM((CH, 3 * D), jnp.float32),
            pltpu.SemaphoreType.DMA,
            pltpu.SemaphoreType.DMA,
            pltpu.SemaphoreType.DMA,
        ],
    )


_NODE_BLK = 2000
_EDGE_BLK = 4000
_INTER_BLK = 2000


def _init_body(x_ref, wi_ref, bi_ref, wm_ref, h_ref, hw_ref):
    h = jnp.maximum(
        jnp.dot(x_ref[...], wi_ref[...], preferred_element_type=jnp.float32)
        + bi_ref[...], 0.0)
    h_ref[...] = h
    hw_ref[...] = jnp.dot(h, wm_ref[...], preferred_element_type=jnp.float32)


def _tc_init(x, w_init, b_init, w_msg0):
    nblk = NG // _NODE_BLK
    return pl.pallas_call(
        _init_body,
        grid=(nblk,),
        in_specs=[
            pl.BlockSpec((_NODE_BLK, D), lambda i: (i, 0)),
            pl.BlockSpec((D, D), lambda i: (0, 0)),
            pl.BlockSpec((1, D), lambda i: (0, 0)),
            pl.BlockSpec((D, D), lambda i: (0, 0)),
        ],
        out_specs=[
            pl.BlockSpec((_NODE_BLK, D), lambda i: (i, 0)),
            pl.BlockSpec((_NODE_BLK, D), lambda i: (i, 0)),
        ],
        out_shape=[
            jax.ShapeDtypeStruct((NG, D), jnp.float32),
            jax.ShapeDtypeStruct((NG, D), jnp.float32),
        ],
    )(x, w_init, b_init.reshape(1, D), w_msg0)


def _edge_proj_body(ea_ref, w_ref, o_ref):
    o_ref[...] = jnp.dot(ea_ref[...], w_ref[...],
                         preferred_element_type=jnp.float32)


def _tc_edge_proj(ea, w):
    n, de = ea.shape
    nblk = n // _EDGE_BLK
    return pl.pallas_call(
        _edge_proj_body,
        grid=(nblk,),
        in_specs=[
            pl.BlockSpec((_EDGE_BLK, de), lambda i: (i, 0)),
            pl.BlockSpec((de, D), lambda i: (0, 0)),
        ],
        out_specs=pl.BlockSpec((_EDGE_BLK, D), lambda i: (i, 0)),
        out_shape=jax.ShapeDtypeStruct((n, D), jnp.float32),
    )(ea, w)


def _layer_body(has_msg, *refs):
    refs = list(refs)
    h_ref = refs.pop(0)
    agg_ref = refs.pop(0)
    ws_ref = refs.pop(0)
    wm_ref = refs.pop(0) if has_msg else None
    ho_ref = refs.pop(0)
    hwo_ref = refs.pop(0) if has_msg else None

    hn = jnp.maximum(
        jnp.dot(h_ref[...], ws_ref[...], preferred_element_type=jnp.float32)
        + agg_ref[...], 0.0)
    ho_ref[...] = hn
    if has_msg:
        hwo_ref[...] = jnp.dot(hn, wm_ref[...],
                               preferred_element_type=jnp.float32)


def _tc_layer(h, agg, w_self, w_msg_next):
    has_msg = w_msg_next is not None
    nblk = NG // _NODE_BLK
    blk = lambda: pl.BlockSpec((_NODE_BLK, D), lambda i: (i, 0))
    wblk = lambda: pl.BlockSpec((D, D), lambda i: (0, 0))
    ins = [h, agg, w_self] + ([w_msg_next] if has_msg else [])
    in_specs = [blk(), blk(), wblk()] + ([wblk()] if has_msg else [])
    nouts = 2 if has_msg else 1
    outs = pl.pallas_call(
        functools.partial(_layer_body, has_msg),
        grid=(nblk,),
        in_specs=in_specs,
        out_specs=[blk() for _ in range(nouts)],
        out_shape=[jax.ShapeDtypeStruct((NG, D), jnp.float32)
                   for _ in range(nouts)],
    )(*ins)
    if has_msg:
        return outs[0], outs[1]
    return outs[0], None


def _inter_body(m_ref, ea_ref, w1_ref, w2_ref, b_ref, s_ref, x_ref):
    i = pl.program_id(0)
    e = jnp.maximum(
        jnp.dot(m_ref[...], w1_ref[...], preferred_element_type=jnp.float32)
        + jnp.dot(ea_ref[...], w2_ref[...], preferred_element_type=jnp.float32)
        + b_ref[...], 0.0)
    sm = jnp.sum(e, axis=0, keepdims=True)
    mx = jnp.max(e, axis=0, keepdims=True)

    @pl.when(i == 0)
    def _():
        s_ref[...] = jnp.zeros_like(s_ref)
        x_ref[...] = jnp.zeros_like(x_ref)

    s_ref[...] += jnp.broadcast_to(sm, (8, D))
    x_ref[...] = jnp.maximum(x_ref[...], jnp.broadcast_to(mx, (8, D)))


def _tc_inter(m, ea, w1, w2, b):
    return pl.pallas_call(
        _inter_body,
        grid=(E // _INTER_BLK,),
        in_specs=[
            pl.BlockSpec((_INTER_BLK, 3 * D), lambda i: (i, 0)),
            pl.BlockSpec((_INTER_BLK, 16), lambda i: (i, 0)),
            pl.BlockSpec((3 * D, D), lambda i: (0, 0)),
            pl.BlockSpec((16, D), lambda i: (0, 0)),
            pl.BlockSpec((1, D), lambda i: (0, 0)),
        ],
        out_specs=[pl.BlockSpec((8, D), lambda i: (0, 0)),
                   pl.BlockSpec((8, D), lambda i: (0, 0))],
        out_shape=[jax.ShapeDtypeStruct((8, D), jnp.float32),
                   jax.ShapeDtypeStruct((8, D), jnp.float32)],
    )(m, ea, w1, w2, b.reshape(1, D))


def _head_body(sp_ref, mp_ref, w1_ref, b1_ref, w2_ref, b2_ref, g_ref, a_ref):
    sm = jnp.sum(sp_ref[...], axis=0, keepdims=True) * (1.0 / (8.0 * E))
    mx = jnp.max(mp_ref[...], axis=0, keepdims=True)
    g = jnp.concatenate([sm, mx], axis=1)
    hfc = jnp.maximum(
        jnp.dot(g, w1_ref[...], preferred_element_type=jnp.float32)
        + b1_ref[...], 0.0)
    aff = jnp.sum(hfc * w2_ref[...], axis=1, keepdims=True) + b2_ref[0, 0]
    g_ref[...] = jnp.broadcast_to(g, (8, 2 * D))
    a_ref[...] = jnp.broadcast_to(aff, (8, D))


def _tc_head(sum_p, max_p, w_fc1, b_fc1, w_fc2, b_fc2):
    full = lambda shape: pl.BlockSpec(shape, lambda: (0, 0))
    return pl.pallas_call(
        _head_body,
        in_specs=[
            full((8, D)), full((8, D)),
            full((2 * D, 2 * D)), full((1, 2 * D)),
            full((1, 2 * D)), full((1, 1)),
        ],
        out_specs=[full((8, 2 * D)), full((8, D))],
        out_shape=[jax.ShapeDtypeStruct((8, 2 * D), jnp.float32),
                   jax.ShapeDtypeStruct((8, D), jnp.float32)],
    )(sum_p, max_p, w_fc1, b_fc1.reshape(1, 2 * D),
      w_fc2.reshape(1, 2 * D), b_fc2.reshape(1, 1))


def kernel(x_lig, edge_index_lig, edge_attr_lig, x_prot, edge_index_prot,
           edge_attr_prot, edge_index_inter, edge_attr_inter, W_init, b_init,
           W_msg_h, W_msg_e, b_msg, W_self, W_inter_m, W_inter_e, b_inter,
           W_fc1, b_fc1, W_fc2, b_fc2):
    x_all = jnp.concatenate([x_lig, x_prot], axis=0)
    src_all = jnp.concatenate(
        [edge_index_lig[0], edge_index_prot[0] + N]).astype(jnp.int32)
    dst_all = jnp.concatenate(
        [edge_index_lig[1], edge_index_prot[1]]).astype(jnp.int32)
    ea_all = jnp.concatenate([edge_attr_lig, edge_attr_prot], axis=0)
    src_i = edge_index_inter[0].astype(jnp.int32)
    dst_i = edge_index_inter[1].astype(jnp.int32)

    h, hw = _tc_init(x_all, W_init, b_init, W_msg_h[0])
    eaw = [_tc_edge_proj(ea_all, W_msg_e[l]) for l in range(3)]

    hs = []
    for l in range(3):
        (agg,) = _sc_layer()(hw, eaw[l], b_msg[l].reshape(1, D),
                             src_all, dst_all)
        w_msg_next = W_msg_h[l + 1] if l < 2 else None
        h, hw = _tc_layer(h, agg, W_self[l], w_msg_next)
        hs.append(h)

    (m,) = _sc_mgather()(hs[0], hs[1], hs[2], src_i, dst_i)
    sum_p, max_p = _tc_inter(m, edge_attr_inter, W_inter_m, W_inter_e,
                             b_inter)

    g8, a8 = _tc_head(sum_p, max_p, W_fc1, b_fc1, W_fc2, b_fc2)
    affinity_pred = a8[0:1, 0:1]
    g = g8[0:1, :]
    ranking = jnp.zeros((1,), jnp.float32)
    return (affinity_pred, g, ranking)


# speedup vs baseline: 2.5356x; 1.1854x over previous
import functools

import jax
import jax.numpy as jnp
from jax import lax
from jax.experimental import pallas as pl
from jax.experimental.pallas import tpu as pltpu
from jax.experimental.pallas import tpu_sc as plsc

N = 10000
NG = 2 * N
E = 160000
D = 128
LANES = 16
NSUB = 16
VPR = D // LANES

CH = 128
EPT = E // NSUB
CHL = 96
NPAIR = EPT // (2 * CHL)
TAIL = EPT - NPAIR * 2 * CHL
RPT = 624

EPT_I = E // (2 * NSUB)
NCH_I = EPT_I // CH
TAIL_I = EPT_I - NCH_I * CH


@functools.cache
def _mesh():
    return plsc.VectorSubcoreMesh(core_axis_name="c", subcore_axis_name="s",
                                  num_cores=2, num_subcores=NSUB)


def _sc_layer_body(hw_hbm, eaw_hbm, b_hbm, src_hbm, dst_hbm, agg_hbm,
                   idx_sa, idx_da, idx_sb, idx_db, idx_st, idx_dt,
                   rows_a, rows_b, eaw_a, eaw_b2, bias_b, acc,
                   sem1, sem2, sem3):
    c = lax.axis_index("c")
    s = lax.axis_index("s")
    zero = jnp.zeros((LANES,), jnp.float32)

    pltpu.sync_copy(b_hbm, bias_b)
    bias = [bias_b[0, pl.ds(k * LANES, LANES)] for k in range(VPR)]

    def zbody(r, carry):
        for k in range(VPR):
            rows_a[r, pl.ds(k * LANES, LANES)] = zero
        return carry

    lax.fori_loop(0, CHL, zbody, 0)
    base_r = s * RPT

    @pl.when(s < NSUB - 1)
    def _():
        for j in range(6):
            pltpu.sync_copy(rows_a, acc.at[pl.ds(base_r + j * CHL, CHL)])
        pltpu.sync_copy(rows_a.at[pl.ds(0, 48)],
                        acc.at[pl.ds(base_r + 6 * CHL, 48)])

    @pl.when(s == NSUB - 1)
    def _():
        for j in range(6):
            pltpu.sync_copy(rows_a, acc.at[pl.ds(base_r + j * CHL, CHL)])
        pltpu.sync_copy(rows_a.at[pl.ds(0, 64)],
                        acc.at[pl.ds(base_r + 6 * CHL, 64)])

    plsc.subcore_barrier()

    def relu_add(rows, eaw, nrows):
        def body(r, carry):
            for k in range(VPR):
                sl = pl.ds(k * LANES, LANES)
                rows[r, sl] = jnp.maximum((rows[r, sl] + eaw[r, sl])
                                          + bias[k], zero)
            return carry
        lax.fori_loop(0, nrows, body, 0)

    tile_base = c * E + s * EPT

    def pair(t, carry):
        a = tile_base + (2 * t) * CHL
        b = a + CHL
        cp_sa = pltpu.async_copy(src_hbm.at[pl.ds(a, CHL)], idx_sa, sem1)
        cp_da = pltpu.async_copy(dst_hbm.at[pl.ds(a, CHL)], idx_da, sem2)
        cp_ea = pltpu.async_copy(eaw_hbm.at[pl.ds(a, CHL)], eaw_a, sem3)
        cp_sb = pltpu.async_copy(src_hbm.at[pl.ds(b, CHL)], idx_sb, sem1)
        cp_db = pltpu.async_copy(dst_hbm.at[pl.ds(b, CHL)], idx_db, sem2)
        cp_eb = pltpu.async_copy(eaw_hbm.at[pl.ds(b, CHL)], eaw_b2, sem3)
        cp_sa.wait()
        g_a = pltpu.async_copy(hw_hbm.at[idx_sa], rows_a, sem1)
        cp_sb.wait()
        g_b = pltpu.async_copy(hw_hbm.at[idx_sb], rows_b, sem2)
        g_a.wait()
        cp_ea.wait()
        relu_add(rows_a, eaw_a, CHL)
        cp_da.wait()
        pltpu.sync_copy(rows_a, acc.at[idx_da], add=True)
        g_b.wait()
        cp_eb.wait()
        relu_add(rows_b, eaw_b2, CHL)
        cp_db.wait()
        pltpu.sync_copy(rows_b, acc.at[idx_db], add=True)
        return carry

    lax.fori_loop(0, NPAIR, pair, 0)

    base = tile_base + NPAIR * 2 * CHL
    cp_s = pltpu.async_copy(src_hbm.at[pl.ds(base, TAIL)], idx_st, sem1)
    cp_d = pltpu.async_copy(dst_hbm.at[pl.ds(base, TAIL)], idx_dt, sem2)
    cp_e = pltpu.async_copy(eaw_hbm.at[pl.ds(base, TAIL)],
                            eaw_a.at[pl.ds(0, TAIL)], sem3)
    cp_s.wait()
    pltpu.async_copy(hw_hbm.at[idx_st], rows_a.at[pl.ds(0, TAIL)],
                     sem1).wait()
    cp_e.wait()
    relu_add(rows_a, eaw_a, TAIL)
    cp_d.wait()
    pltpu.sync_copy(rows_a.at[pl.ds(0, TAIL)], acc.at[idx_dt], add=True)

    plsc.subcore_barrier()
    out_base = c * N + base_r

    @pl.when(s < NSUB - 1)
    def _():
        pltpu.sync_copy(acc.at[pl.ds(base_r, RPT)],
                        agg_hbm.at[pl.ds(out_base, RPT)])

    @pl.when(s == NSUB - 1)
    def _():
        pltpu.sync_copy(acc.at[pl.ds(base_r, 640)],
                        agg_hbm.at[pl.ds(out_base, 640)])


@functools.cache
def _sc_layer():
    return pl.kernel(
        _sc_layer_body,
        out_type=[jax.ShapeDtypeStruct((NG, D), jnp.float32)],
        mesh=_mesh(),
        scratch_types=[
            pltpu.VMEM((CHL,), jnp.int32),
            pltpu.VMEM((CHL,), jnp.int32),
            pltpu.VMEM((CHL,), jnp.int32),
            pltpu.VMEM((CHL,), jnp.int32),
            pltpu.VMEM((TAIL,), jnp.int32),
            pltpu.VMEM((TAIL,), jnp.int32),
            pltpu.VMEM((CHL, D), jnp.float32),
            pltpu.VMEM((CHL, D), jnp.float32),
            pltpu.VMEM((CHL, D), jnp.float32),
            pltpu.VMEM((CHL, D), jnp.float32),
            pltpu.VMEM((1, D), jnp.float32),
            pltpu.VMEM_SHARED((N, D), jnp.float32),
            pltpu.SemaphoreType.DMA,
            pltpu.SemaphoreType.DMA,
            pltpu.SemaphoreType.DMA,
        ],
    )


def _sc_mgather_body(h1_hbm, h2_hbm, h3_hbm, src_hbm, dst_hbm, m_hbm,
                     idx_s, idx_d, idx_st, idx_dt, g1, g2,
                     sem1, sem2, sem3):
    c = lax.axis_index("c")
    s = lax.axis_index("s")
    wid = c * NSUB + s
    tile_base = wid * EPT_I
    tabs = (h1_hbm, h2_hbm, h3_hbm)

    def gather_add(nrows, isrc, idst, base):
        cps = [pltpu.async_copy(tab.at[isrc], g1.at[t].at[pl.ds(0, nrows)],
                                sem1) for t, tab in enumerate(tabs)]
        cpd = [pltpu.async_copy(tab.at[idst], g2.at[t].at[pl.ds(0, nrows)],
                                sem2) for t, tab in enumerate(tabs)]
        for cp in cps:
            cp.wait()
        for cp in cpd:
            cp.wait()

        def rbody(r, carry):
            for t in range(3):
                for k in range(VPR):
                    sl = pl.ds(k * LANES, LANES)
                    g2[t, r, sl] = g1[t, r, sl] + g2[t, r, sl]
            return carry
        lax.fori_loop(0, nrows, rbody, 0)
        wr = [pltpu.async_copy(g2.at[t].at[pl.ds(0, nrows)],
                               m_hbm.at[t].at[pl.ds(base, nrows)], sem3)
              for t in range(3)]
        for cp in wr:
            cp.wait()

    def chunk(i, carry):
        base = tile_base + i * CH
        cp_s = pltpu.async_copy(src_hbm.at[pl.ds(base, CH)], idx_s, sem1)
        cp_d = pltpu.async_copy(dst_hbm.at[pl.ds(base, CH)], idx_d, sem2)
        cp_s.wait()
        cp_d.wait()
        gather_add(CH, idx_s, idx_d, base)
        return carry

    lax.fori_loop(0, NCH_I, chunk, 0)

    base = tile_base + NCH_I * CH
    cp_s = pltpu.async_copy(src_hbm.at[pl.ds(base, TAIL_I)], idx_st, sem1)
    cp_d = pltpu.async_copy(dst_hbm.at[pl.ds(base, TAIL_I)], idx_dt, sem2)
    cp_s.wait()
    cp_d.wait()
    gather_add(TAIL_I, idx_st, idx_dt, base)


@functools.cache
def _sc_mgather():
    return pl.kernel(
        _sc_mgather_body,
        out_type=[jax.ShapeDtypeStruct((3, E, D), jnp.float32)],
        mesh=_mesh(),
        scratch_types=[
            pltpu.VMEM((CH,), jnp.int32),
            pltpu.VMEM((CH,), jnp.int32),
            pltpu.VMEM((TAIL_I,), jnp.int32),
            pltpu.VMEM((TAIL_I,), jnp.int32),
            pltpu.VMEM((3, CH, D), jnp.float32),
            pltpu.VMEM((3, CH, D), jnp.float32),
            pltpu.SemaphoreType.DMA,
            pltpu.SemaphoreType.DMA,
            pltpu.SemaphoreType.DMA,
        ],
    )


_NODE_BLK = 2000
_EDGE_BLK = 4000
_INTER_BLK = 2000


def _init_body(x_ref, wi_ref, bi_ref, wm_ref, h_ref, hw_ref):
    h = jnp.maximum(
        jnp.dot(x_ref[...], wi_ref[...], preferred_element_type=jnp.float32)
        + bi_ref[...], 0.0)
    h_ref[...] = h
    hw_ref[...] = jnp.dot(h, wm_ref[...], preferred_element_type=jnp.float32)


def _tc_init(x, w_init, b_init, w_msg0):
    nblk = NG // _NODE_BLK
    return pl.pallas_call(
        _init_body,
        grid=(nblk,),
        in_specs=[
            pl.BlockSpec((_NODE_BLK, D), lambda i: (i, 0)),
            pl.BlockSpec((D, D), lambda i: (0, 0)),
            pl.BlockSpec((1, D), lambda i: (0, 0)),
            pl.BlockSpec((D, D), lambda i: (0, 0)),
        ],
        out_specs=[
            pl.BlockSpec((_NODE_BLK, D), lambda i: (i, 0)),
            pl.BlockSpec((_NODE_BLK, D), lambda i: (i, 0)),
        ],
        out_shape=[
            jax.ShapeDtypeStruct((NG, D), jnp.float32),
            jax.ShapeDtypeStruct((NG, D), jnp.float32),
        ],
    )(x, w_init, b_init.reshape(1, D), w_msg0)


def _edge_proj_body(ea_ref, w_ref, o0_ref, o1_ref, o2_ref):
    ea = ea_ref[...]
    w = w_ref[...]
    o0_ref[...] = jnp.dot(ea, w[0:16, :], preferred_element_type=jnp.float32)
    o1_ref[...] = jnp.dot(ea, w[16:32, :], preferred_element_type=jnp.float32)
    o2_ref[...] = jnp.dot(ea, w[32:48, :], preferred_element_type=jnp.float32)


def _tc_edge_proj3(ea, w3):
    n = ea.shape[0]
    nblk = n // _EDGE_BLK
    oblk = lambda: pl.BlockSpec((_EDGE_BLK, D), lambda i: (i, 0))
    return pl.pallas_call(
        _edge_proj_body,
        grid=(nblk,),
        in_specs=[
            pl.BlockSpec((_EDGE_BLK, 16), lambda i: (i, 0)),
            pl.BlockSpec((48, D), lambda i: (0, 0)),
        ],
        out_specs=[oblk(), oblk(), oblk()],
        out_shape=[jax.ShapeDtypeStruct((n, D), jnp.float32)
                   for _ in range(3)],
    )(ea, w3)


def _layer_body(has_msg, *refs):
    refs = list(refs)
    h_ref = refs.pop(0)
    agg_ref = refs.pop(0)
    ws_ref = refs.pop(0)
    wm_ref = refs.pop(0) if has_msg else None
    ho_ref = refs.pop(0)
    hwo_ref = refs.pop(0) if has_msg else None

    hn = jnp.maximum(
        jnp.dot(h_ref[...], ws_ref[...], preferred_element_type=jnp.float32)
        + agg_ref[...], 0.0)
    ho_ref[...] = hn
    if has_msg:
        hwo_ref[...] = jnp.dot(hn, wm_ref[...],
                               preferred_element_type=jnp.float32)


def _tc_layer(h, agg, w_self, w_msg_next):
    has_msg = w_msg_next is not None
    nblk = NG // _NODE_BLK
    blk = lambda: pl.BlockSpec((_NODE_BLK, D), lambda i: (i, 0))
    wblk = lambda: pl.BlockSpec((D, D), lambda i: (0, 0))
    ins = [h, agg, w_self] + ([w_msg_next] if has_msg else [])
    in_specs = [blk(), blk(), wblk()] + ([wblk()] if has_msg else [])
    nouts = 2 if has_msg else 1
    outs = pl.pallas_call(
        functools.partial(_layer_body, has_msg),
        grid=(nblk,),
        in_specs=in_specs,
        out_specs=[blk() for _ in range(nouts)],
        out_shape=[jax.ShapeDtypeStruct((NG, D), jnp.float32)
                   for _ in range(nouts)],
    )(*ins)
    if has_msg:
        return outs[0], outs[1]
    return outs[0], None


def _inter_body(m1_ref, m2_ref, m3_ref, ea_ref, w1_ref, w2_ref, b_ref,
                s_ref, x_ref):
    i = pl.program_id(0)
    w1 = w1_ref[...]
    mw = (jnp.dot(m1_ref[...], w1[0:D, :],
                  preferred_element_type=jnp.float32)
          + jnp.dot(m2_ref[...], w1[D:2 * D, :],
                    preferred_element_type=jnp.float32)
          + jnp.dot(m3_ref[...], w1[2 * D:3 * D, :],
                    preferred_element_type=jnp.float32))
    e = jnp.maximum(
        mw
        + jnp.dot(ea_ref[...], w2_ref[...], preferred_element_type=jnp.float32)
        + b_ref[...], 0.0)
    sm = jnp.sum(e, axis=0, keepdims=True)
    mx = jnp.max(e, axis=0, keepdims=True)

    @pl.when(i == 0)
    def _():
        s_ref[...] = jnp.zeros_like(s_ref)
        x_ref[...] = jnp.zeros_like(x_ref)

    s_ref[...] += jnp.broadcast_to(sm, (8, D))
    x_ref[...] = jnp.maximum(x_ref[...], jnp.broadcast_to(mx, (8, D)))


def _tc_inter(m1, m2, m3, ea, w1, w2, b):
    eblk = lambda: pl.BlockSpec((_INTER_BLK, D), lambda i: (i, 0))
    return pl.pallas_call(
        _inter_body,
        grid=(E // _INTER_BLK,),
        in_specs=[
            eblk(), eblk(), eblk(),
            pl.BlockSpec((_INTER_BLK, 16), lambda i: (i, 0)),
            pl.BlockSpec((3 * D, D), lambda i: (0, 0)),
            pl.BlockSpec((16, D), lambda i: (0, 0)),
            pl.BlockSpec((1, D), lambda i: (0, 0)),
        ],
        out_specs=[pl.BlockSpec((8, D), lambda i: (0, 0)),
                   pl.BlockSpec((8, D), lambda i: (0, 0))],
        out_shape=[jax.ShapeDtypeStruct((8, D), jnp.float32),
                   jax.ShapeDtypeStruct((8, D), jnp.float32)],
    )(m1, m2, m3, ea, w1, w2, b.reshape(1, D))


def _head_body(sp_ref, mp_ref, w1_ref, b1_ref, w2_ref, b2_ref, g_ref, a_ref):
    sm = jnp.sum(sp_ref[...], axis=0, keepdims=True) * (1.0 / (8.0 * E))
    mx = jnp.max(mp_ref[...], axis=0, keepdims=True)
    g = jnp.concatenate([sm, mx], axis=1)
    hfc = jnp.maximum(
        jnp.dot(g, w1_ref[...], preferred_element_type=jnp.float32)
        + b1_ref[...], 0.0)
    aff = jnp.sum(hfc * w2_ref[...], axis=1, keepdims=True) + b2_ref[0, 0]
    g_ref[...] = jnp.broadcast_to(g, (8, 2 * D))
    a_ref[...] = jnp.broadcast_to(aff, (8, D))


def _tc_head(sum_p, max_p, w_fc1, b_fc1, w_fc2, b_fc2):
    full = lambda shape: pl.BlockSpec(shape, lambda: (0, 0))
    return pl.pallas_call(
        _head_body,
        in_specs=[
            full((8, D)), full((8, D)),
            full((2 * D, 2 * D)), full((1, 2 * D)),
            full((1, 2 * D)), full((1, 1)),
        ],
        out_specs=[full((8, 2 * D)), full((8, D))],
        out_shape=[jax.ShapeDtypeStruct((8, 2 * D), jnp.float32),
                   jax.ShapeDtypeStruct((8, D), jnp.float32)],
    )(sum_p, max_p, w_fc1, b_fc1.reshape(1, 2 * D),
      w_fc2.reshape(1, 2 * D), b_fc2.reshape(1, 1))


def kernel(x_lig, edge_index_lig, edge_attr_lig, x_prot, edge_index_prot,
           edge_attr_prot, edge_index_inter, edge_attr_inter, W_init, b_init,
           W_msg_h, W_msg_e, b_msg, W_self, W_inter_m, W_inter_e, b_inter,
           W_fc1, b_fc1, W_fc2, b_fc2):
    x_all = jnp.concatenate([x_lig, x_prot], axis=0)
    src_all = jnp.concatenate(
        [edge_index_lig[0], edge_index_prot[0] + N]).astype(jnp.int32)
    dst_all = jnp.concatenate(
        [edge_index_lig[1], edge_index_prot[1]]).astype(jnp.int32)
    ea_all = jnp.concatenate([edge_attr_lig, edge_attr_prot], axis=0)
    src_i = edge_index_inter[0].astype(jnp.int32)
    dst_i = edge_index_inter[1].astype(jnp.int32)

    h, hw = _tc_init(x_all, W_init, b_init, W_msg_h[0])
    eaw = _tc_edge_proj3(ea_all, W_msg_e.reshape(48, D))

    hs = []
    for l in range(3):
        (agg,) = _sc_layer()(hw, eaw[l], b_msg[l].reshape(1, D),
                             src_all, dst_all)
        w_msg_next = W_msg_h[l + 1] if l < 2 else None
        h, hw = _tc_layer(h, agg, W_self[l], w_msg_next)
        hs.append(h)

    (m,) = _sc_mgather()(hs[0], hs[1], hs[2], src_i, dst_i)
    sum_p, max_p = _tc_inter(m[0], m[1], m[2], edge_attr_inter, W_inter_m,
                             W_inter_e, b_inter)

    g8, a8 = _tc_head(sum_p, max_p, W_fc1, b_fc1, W_fc2, b_fc2)
    affinity_pred = a8[0:1, 0:1]
    g = g8[0:1, :]
    ranking = jnp.zeros((1,), jnp.float32)
    return (affinity_pred, g, ranking)


# speedup vs baseline: 2.5869x; 1.0202x over previous
import functools

import jax
import jax.numpy as jnp
from jax import lax
from jax.experimental import pallas as pl
from jax.experimental.pallas import tpu as pltpu
from jax.experimental.pallas import tpu_sc as plsc

N = 10000
NG = 2 * N
E = 160000
D = 128
LANES = 16
NSUB = 16
VPR = D // LANES

CH = 128
EPT = E // NSUB
CHL = 96
NPAIR = EPT // (2 * CHL)
TAIL = EPT - NPAIR * 2 * CHL
RPT = 624

EPT_I = E // (2 * NSUB)
CHM = 64
NPAIR_I = EPT_I // (2 * CHM)
TAIL_I = EPT_I - NPAIR_I * 2 * CHM


@functools.cache
def _mesh():
    return plsc.VectorSubcoreMesh(core_axis_name="c", subcore_axis_name="s",
                                  num_cores=2, num_subcores=NSUB)


def _sc_layer_body(hw_hbm, eaw_hbm, b_hbm, src_hbm, dst_hbm, agg_hbm,
                   idx_sa, idx_da, idx_sb, idx_db, idx_st, idx_dt,
                   rows_a, rows_b, eaw_a, eaw_b2, bias_b, acc,
                   sem1, sem2, sem3):
    c = lax.axis_index("c")
    s = lax.axis_index("s")
    zero = jnp.zeros((LANES,), jnp.float32)

    pltpu.sync_copy(b_hbm, bias_b)
    bias = [bias_b[0, pl.ds(k * LANES, LANES)] for k in range(VPR)]

    def zbody(r, carry):
        for k in range(VPR):
            rows_a[r, pl.ds(k * LANES, LANES)] = zero
        return carry

    lax.fori_loop(0, CHL, zbody, 0)
    base_r = s * RPT

    @pl.when(s < NSUB - 1)
    def _():
        for j in range(6):
            pltpu.sync_copy(rows_a, acc.at[pl.ds(base_r + j * CHL, CHL)])
        pltpu.sync_copy(rows_a.at[pl.ds(0, 48)],
                        acc.at[pl.ds(base_r + 6 * CHL, 48)])

    @pl.when(s == NSUB - 1)
    def _():
        for j in range(6):
            pltpu.sync_copy(rows_a, acc.at[pl.ds(base_r + j * CHL, CHL)])
        pltpu.sync_copy(rows_a.at[pl.ds(0, 64)],
                        acc.at[pl.ds(base_r + 6 * CHL, 64)])

    plsc.subcore_barrier()

    def relu_add(rows, eaw, nrows):
        def body(r, carry):
            for k in range(VPR):
                sl = pl.ds(k * LANES, LANES)
                rows[r, sl] = jnp.maximum((rows[r, sl] + eaw[r, sl])
                                          + bias[k], zero)
            return carry
        lax.fori_loop(0, nrows, body, 0)

    tile_base = c * E + s * EPT

    def pair(t, carry):
        a = tile_base + (2 * t) * CHL
        b = a + CHL
        cp_sa = pltpu.async_copy(src_hbm.at[pl.ds(a, CHL)], idx_sa, sem1)
        cp_da = pltpu.async_copy(dst_hbm.at[pl.ds(a, CHL)], idx_da, sem2)
        cp_ea = pltpu.async_copy(eaw_hbm.at[pl.ds(a, CHL)], eaw_a, sem3)
        cp_sb = pltpu.async_copy(src_hbm.at[pl.ds(b, CHL)], idx_sb, sem1)
        cp_db = pltpu.async_copy(dst_hbm.at[pl.ds(b, CHL)], idx_db, sem2)
        cp_eb = pltpu.async_copy(eaw_hbm.at[pl.ds(b, CHL)], eaw_b2, sem3)
        cp_sa.wait()
        g_a = pltpu.async_copy(hw_hbm.at[idx_sa], rows_a, sem1)
        cp_sb.wait()
        g_b = pltpu.async_copy(hw_hbm.at[idx_sb], rows_b, sem2)
        g_a.wait()
        cp_ea.wait()
        relu_add(rows_a, eaw_a, CHL)
        cp_da.wait()
        pltpu.sync_copy(rows_a, acc.at[idx_da], add=True)
        g_b.wait()
        cp_eb.wait()
        relu_add(rows_b, eaw_b2, CHL)
        cp_db.wait()
        pltpu.sync_copy(rows_b, acc.at[idx_db], add=True)
        return carry

    lax.fori_loop(0, NPAIR, pair, 0)

    base = tile_base + NPAIR * 2 * CHL
    cp_s = pltpu.async_copy(src_hbm.at[pl.ds(base, TAIL)], idx_st, sem1)
    cp_d = pltpu.async_copy(dst_hbm.at[pl.ds(base, TAIL)], idx_dt, sem2)
    cp_e = pltpu.async_copy(eaw_hbm.at[pl.ds(base, TAIL)],
                            eaw_a.at[pl.ds(0, TAIL)], sem3)
    cp_s.wait()
    pltpu.async_copy(hw_hbm.at[idx_st], rows_a.at[pl.ds(0, TAIL)],
                     sem1).wait()
    cp_e.wait()
    relu_add(rows_a, eaw_a, TAIL)
    cp_d.wait()
    pltpu.sync_copy(rows_a.at[pl.ds(0, TAIL)], acc.at[idx_dt], add=True)

    plsc.subcore_barrier()
    out_base = c * N + base_r

    @pl.when(s < NSUB - 1)
    def _():
        pltpu.sync_copy(acc.at[pl.ds(base_r, RPT)],
                        agg_hbm.at[pl.ds(out_base, RPT)])

    @pl.when(s == NSUB - 1)
    def _():
        pltpu.sync_copy(acc.at[pl.ds(base_r, 640)],
                        agg_hbm.at[pl.ds(out_base, 640)])


@functools.cache
def _sc_layer():
    return pl.kernel(
        _sc_layer_body,
        out_type=[jax.ShapeDtypeStruct((NG, D), jnp.float32)],
        mesh=_mesh(),
        scratch_types=[
            pltpu.VMEM((CHL,), jnp.int32),
            pltpu.VMEM((CHL,), jnp.int32),
            pltpu.VMEM((CHL,), jnp.int32),
            pltpu.VMEM((CHL,), jnp.int32),
            pltpu.VMEM((TAIL,), jnp.int32),
            pltpu.VMEM((TAIL,), jnp.int32),
            pltpu.VMEM((CHL, D), jnp.float32),
            pltpu.VMEM((CHL, D), jnp.float32),
            pltpu.VMEM((CHL, D), jnp.float32),
            pltpu.VMEM((CHL, D), jnp.float32),
            pltpu.VMEM((1, D), jnp.float32),
            pltpu.VMEM_SHARED((N, D), jnp.float32),
            pltpu.SemaphoreType.DMA,
            pltpu.SemaphoreType.DMA,
            pltpu.SemaphoreType.DMA,
        ],
    )


def _sc_mgather_body(h1_hbm, h2_hbm, h3_hbm, src_hbm, dst_hbm, m_hbm,
                     idx_sa, idx_da, idx_sb, idx_db, idx_st, idx_dt,
                     g1a, g2a, g1b, g2b, sem1, sem2, sem3):
    c = lax.axis_index("c")
    s = lax.axis_index("s")
    wid = c * NSUB + s
    tile_base = wid * EPT_I
    tabs = (h1_hbm, h2_hbm, h3_hbm)

    def add_rows(g1, g2, nrows):
        def rbody(r, carry):
            for t in range(3):
                for k in range(VPR):
                    sl = pl.ds(k * LANES, LANES)
                    g2[t, r, sl] = g1[t, r, sl] + g2[t, r, sl]
            return carry
        lax.fori_loop(0, nrows, rbody, 0)

    def drain_writes():
        for t in range(3):
            pltpu.make_async_copy(m_hbm.at[t].at[pl.ds(0, CHM)],
                                  g2a.at[t], sem3).wait()
            pltpu.make_async_copy(m_hbm.at[t].at[pl.ds(0, CHM)],
                                  g2b.at[t], sem3).wait()

    def pair(i, carry):
        a = tile_base + (2 * i) * CHM
        b = a + CHM

        @pl.when(i > 0)
        def _():
            drain_writes()

        cp_sa = pltpu.async_copy(src_hbm.at[pl.ds(a, CHM)], idx_sa, sem1)
        cp_da = pltpu.async_copy(dst_hbm.at[pl.ds(a, CHM)], idx_da, sem1)
        cp_sb = pltpu.async_copy(src_hbm.at[pl.ds(b, CHM)], idx_sb, sem2)
        cp_db = pltpu.async_copy(dst_hbm.at[pl.ds(b, CHM)], idx_db, sem2)
        cp_sa.wait()
        cp_da.wait()
        ga = ([pltpu.async_copy(tab.at[idx_sa], g1a.at[t], sem1)
               for t, tab in enumerate(tabs)]
              + [pltpu.async_copy(tab.at[idx_da], g2a.at[t], sem1)
                 for t, tab in enumerate(tabs)])
        cp_sb.wait()
        cp_db.wait()
        gb = ([pltpu.async_copy(tab.at[idx_sb], g1b.at[t], sem2)
               for t, tab in enumerate(tabs)]
              + [pltpu.async_copy(tab.at[idx_db], g2b.at[t], sem2)
                 for t, tab in enumerate(tabs)])
        for cp in ga:
            cp.wait()
        add_rows(g1a, g2a, CHM)
        for t in range(3):
            pltpu.async_copy(g2a.at[t], m_hbm.at[t].at[pl.ds(a, CHM)], sem3)
        for cp in gb:
            cp.wait()
        add_rows(g1b, g2b, CHM)
        for t in range(3):
            pltpu.async_copy(g2b.at[t], m_hbm.at[t].at[pl.ds(b, CHM)], sem3)
        return carry

    lax.fori_loop(0, NPAIR_I, pair, 0)
    drain_writes()

    base = tile_base + NPAIR_I * 2 * CHM
    cp_s = pltpu.async_copy(src_hbm.at[pl.ds(base, TAIL_I)], idx_st, sem1)
    cp_d = pltpu.async_copy(dst_hbm.at[pl.ds(base, TAIL_I)], idx_dt, sem2)
    cp_s.wait()
    cp_d.wait()
    gt = ([pltpu.async_copy(tab.at[idx_st], g1a.at[t].at[pl.ds(0, TAIL_I)],
                            sem1) for t, tab in enumerate(tabs)]
          + [pltpu.async_copy(tab.at[idx_dt], g2a.at[t].at[pl.ds(0, TAIL_I)],
                              sem2) for t, tab in enumerate(tabs)])
    for cp in gt:
        cp.wait()
    add_rows(g1a, g2a, TAIL_I)
    for t in range(3):
        pltpu.sync_copy(g2a.at[t].at[pl.ds(0, TAIL_I)],
                        m_hbm.at[t].at[pl.ds(base, TAIL_I)])


@functools.cache
def _sc_mgather():
    return pl.kernel(
        _sc_mgather_body,
        out_type=[jax.ShapeDtypeStruct((3, E, D), jnp.float32)],
        mesh=_mesh(),
        scratch_types=[
            pltpu.VMEM((CHM,), jnp.int32),
            pltpu.VMEM((CHM,), jnp.int32),
            pltpu.VMEM((CHM,), jnp.int32),
            pltpu.VMEM((CHM,), jnp.int32),
            pltpu.VMEM((TAIL_I,), jnp.int32),
            pltpu.VMEM((TAIL_I,), jnp.int32),
            pltpu.VMEM((3, CHM, D), jnp.float32),
            pltpu.VMEM((3, CHM, D), jnp.float32),
            pltpu.VMEM((3, CHM, D), jnp.float32),
            pltpu.VMEM((3, CHM, D), jnp.float32),
            pltpu.SemaphoreType.DMA,
            pltpu.SemaphoreType.DMA,
            pltpu.SemaphoreType.DMA,
        ],
    )


_NODE_BLK = 2000
_EDGE_BLK = 4000
_INTER_BLK = 2000


def _init_body(x_ref, wi_ref, bi_ref, wm_ref, h_ref, hw_ref):
    h = jnp.maximum(
        jnp.dot(x_ref[...], wi_ref[...], preferred_element_type=jnp.float32)
        + bi_ref[...], 0.0)
    h_ref[...] = h
    hw_ref[...] = jnp.dot(h, wm_ref[...], preferred_element_type=jnp.float32)


def _tc_init(x, w_init, b_init, w_msg0):
    nblk = NG // _NODE_BLK
    return pl.pallas_call(
        _init_body,
        grid=(nblk,),
        in_specs=[
            pl.BlockSpec((_NODE_BLK, D), lambda i: (i, 0)),
            pl.BlockSpec((D, D), lambda i: (0, 0)),
            pl.BlockSpec((1, D), lambda i: (0, 0)),
            pl.BlockSpec((D, D), lambda i: (0, 0)),
        ],
        out_specs=[
            pl.BlockSpec((_NODE_BLK, D), lambda i: (i, 0)),
            pl.BlockSpec((_NODE_BLK, D), lambda i: (i, 0)),
        ],
        out_shape=[
            jax.ShapeDtypeStruct((NG, D), jnp.float32),
            jax.ShapeDtypeStruct((NG, D), jnp.float32),
        ],
    )(x, w_init, b_init.reshape(1, D), w_msg0)


def _edge_proj_body(ea_ref, w_ref, o0_ref, o1_ref, o2_ref):
    ea = ea_ref[...]
    w = w_ref[...]
    o0_ref[...] = jnp.dot(ea, w[0:16, :], preferred_element_type=jnp.float32)
    o1_ref[...] = jnp.dot(ea, w[16:32, :], preferred_element_type=jnp.float32)
    o2_ref[...] = jnp.dot(ea, w[32:48, :], preferred_element_type=jnp.float32)


def _tc_edge_proj3(ea, w3):
    n = ea.shape[0]
    nblk = n // _EDGE_BLK
    oblk = lambda: pl.BlockSpec((_EDGE_BLK, D), lambda i: (i, 0))
    return pl.pallas_call(
        _edge_proj_body,
        grid=(nblk,),
        in_specs=[
            pl.BlockSpec((_EDGE_BLK, 16), lambda i: (i, 0)),
            pl.BlockSpec((48, D), lambda i: (0, 0)),
        ],
        out_specs=[oblk(), oblk(), oblk()],
        out_shape=[jax.ShapeDtypeStruct((n, D), jnp.float32)
                   for _ in range(3)],
    )(ea, w3)


def _layer_body(has_msg, *refs):
    refs = list(refs)
    h_ref = refs.pop(0)
    agg_ref = refs.pop(0)
    ws_ref = refs.pop(0)
    wm_ref = refs.pop(0) if has_msg else None
    ho_ref = refs.pop(0)
    hwo_ref = refs.pop(0) if has_msg else None

    hn = jnp.maximum(
        jnp.dot(h_ref[...], ws_ref[...], preferred_element_type=jnp.float32)
        + agg_ref[...], 0.0)
    ho_ref[...] = hn
    if has_msg:
        hwo_ref[...] = jnp.dot(hn, wm_ref[...],
                               preferred_element_type=jnp.float32)


def _tc_layer(h, agg, w_self, w_msg_next):
    has_msg = w_msg_next is not None
    nblk = NG // _NODE_BLK
    blk = lambda: pl.BlockSpec((_NODE_BLK, D), lambda i: (i, 0))
    wblk = lambda: pl.BlockSpec((D, D), lambda i: (0, 0))
    ins = [h, agg, w_self] + ([w_msg_next] if has_msg else [])
    in_specs = [blk(), blk(), wblk()] + ([wblk()] if has_msg else [])
    nouts = 2 if has_msg else 1
    outs = pl.pallas_call(
        functools.partial(_layer_body, has_msg),
        grid=(nblk,),
        in_specs=in_specs,
        out_specs=[blk() for _ in range(nouts)],
        out_shape=[jax.ShapeDtypeStruct((NG, D), jnp.float32)
                   for _ in range(nouts)],
    )(*ins)
    if has_msg:
        return outs[0], outs[1]
    return outs[0], None


def _inter_body(m1_ref, m2_ref, m3_ref, ea_ref, w1_ref, w2_ref, b_ref,
                s_ref, x_ref):
    i = pl.program_id(0)
    w1 = w1_ref[...]
    mw = (jnp.dot(m1_ref[...], w1[0:D, :],
                  preferred_element_type=jnp.float32)
          + jnp.dot(m2_ref[...], w1[D:2 * D, :],
                    preferred_element_type=jnp.float32)
          + jnp.dot(m3_ref[...], w1[2 * D:3 * D, :],
                    preferred_element_type=jnp.float32))
    e = jnp.maximum(
        mw
        + jnp.dot(ea_ref[...], w2_ref[...], preferred_element_type=jnp.float32)
        + b_ref[...], 0.0)
    sm = jnp.sum(e, axis=0, keepdims=True)
    mx = jnp.max(e, axis=0, keepdims=True)

    @pl.when(i == 0)
    def _():
        s_ref[...] = jnp.zeros_like(s_ref)
        x_ref[...] = jnp.zeros_like(x_ref)

    s_ref[...] += jnp.broadcast_to(sm, (8, D))
    x_ref[...] = jnp.maximum(x_ref[...], jnp.broadcast_to(mx, (8, D)))


def _tc_inter(m1, m2, m3, ea, w1, w2, b):
    eblk = lambda: pl.BlockSpec((_INTER_BLK, D), lambda i: (i, 0))
    return pl.pallas_call(
        _inter_body,
        grid=(E // _INTER_BLK,),
        in_specs=[
            eblk(), eblk(), eblk(),
            pl.BlockSpec((_INTER_BLK, 16), lambda i: (i, 0)),
            pl.BlockSpec((3 * D, D), lambda i: (0, 0)),
            pl.BlockSpec((16, D), lambda i: (0, 0)),
            pl.BlockSpec((1, D), lambda i: (0, 0)),
        ],
        out_specs=[pl.BlockSpec((8, D), lambda i: (0, 0)),
                   pl.BlockSpec((8, D), lambda i: (0, 0))],
        out_shape=[jax.ShapeDtypeStruct((8, D), jnp.float32),
                   jax.ShapeDtypeStruct((8, D), jnp.float32)],
    )(m1, m2, m3, ea, w1, w2, b.reshape(1, D))


def _head_body(sp_ref, mp_ref, w1_ref, b1_ref, w2_ref, b2_ref, g_ref, a_ref):
    sm = jnp.sum(sp_ref[...], axis=0, keepdims=True) * (1.0 / (8.0 * E))
    mx = jnp.max(mp_ref[...], axis=0, keepdims=True)
    g = jnp.concatenate([sm, mx], axis=1)
    hfc = jnp.maximum(
        jnp.dot(g, w1_ref[...], preferred_element_type=jnp.float32)
        + b1_ref[...], 0.0)
    aff = jnp.sum(hfc * w2_ref[...], axis=1, keepdims=True) + b2_ref[0, 0]
    g_ref[...] = jnp.broadcast_to(g, (8, 2 * D))
    a_ref[...] = jnp.broadcast_to(aff, (8, D))


def _tc_head(sum_p, max_p, w_fc1, b_fc1, w_fc2, b_fc2):
    full = lambda shape: pl.BlockSpec(shape, lambda: (0, 0))
    return pl.pallas_call(
        _head_body,
        in_specs=[
            full((8, D)), full((8, D)),
            full((2 * D, 2 * D)), full((1, 2 * D)),
            full((1, 2 * D)), full((1, 1)),
        ],
        out_specs=[full((8, 2 * D)), full((8, D))],
        out_shape=[jax.ShapeDtypeStruct((8, 2 * D), jnp.float32),
                   jax.ShapeDtypeStruct((8, D), jnp.float32)],
    )(sum_p, max_p, w_fc1, b_fc1.reshape(1, 2 * D),
      w_fc2.reshape(1, 2 * D), b_fc2.reshape(1, 1))


def kernel(x_lig, edge_index_lig, edge_attr_lig, x_prot, edge_index_prot,
           edge_attr_prot, edge_index_inter, edge_attr_inter, W_init, b_init,
           W_msg_h, W_msg_e, b_msg, W_self, W_inter_m, W_inter_e, b_inter,
           W_fc1, b_fc1, W_fc2, b_fc2):
    x_all = jnp.concatenate([x_lig, x_prot], axis=0)
    src_all = jnp.concatenate(
        [edge_index_lig[0], edge_index_prot[0] + N]).astype(jnp.int32)
    dst_all = jnp.concatenate(
        [edge_index_lig[1], edge_index_prot[1]]).astype(jnp.int32)
    ea_all = jnp.concatenate([edge_attr_lig, edge_attr_prot], axis=0)
    src_i = edge_index_inter[0].astype(jnp.int32)
    dst_i = edge_index_inter[1].astype(jnp.int32)

    h, hw = _tc_init(x_all, W_init, b_init, W_msg_h[0])
    eaw = _tc_edge_proj3(ea_all, W_msg_e.reshape(48, D))

    hs = []
    for l in range(3):
        (agg,) = _sc_layer()(hw, eaw[l], b_msg[l].reshape(1, D),
                             src_all, dst_all)
        w_msg_next = W_msg_h[l + 1] if l < 2 else None
        h, hw = _tc_layer(h, agg, W_self[l], w_msg_next)
        hs.append(h)

    (m,) = _sc_mgather()(hs[0], hs[1], hs[2], src_i, dst_i)
    sum_p, max_p = _tc_inter(m[0], m[1], m[2], edge_attr_inter, W_inter_m,
                             W_inter_e, b_inter)

    g8, a8 = _tc_head(sum_p, max_p, W_fc1, b_fc1, W_fc2, b_fc2)
    affinity_pred = a8[0:1, 0:1]
    g = g8[0:1, :]
    ranking = jnp.zeros((1,), jnp.float32)
    return (affinity_pred, g, ranking)


# speedup vs baseline: 2.6870x; 1.0387x over previous
import functools

import jax
import jax.numpy as jnp
from jax import lax
from jax.experimental import pallas as pl
from jax.experimental.pallas import tpu as pltpu
from jax.experimental.pallas import tpu_sc as plsc

N = 10000
NG = 2 * N
E = 160000
D = 128
LANES = 16
NSUB = 16
VPR = D // LANES

CH = 128
EPT = E // NSUB
CHL = 96
NPAIR = EPT // (2 * CHL)
TAIL = EPT - NPAIR * 2 * CHL
RPT = 624

EPT_I = E // (2 * NSUB)
CHM = 64
NPAIR_I = EPT_I // (2 * CHM)
TAIL_I = EPT_I - NPAIR_I * 2 * CHM


@functools.cache
def _mesh():
    return plsc.VectorSubcoreMesh(core_axis_name="c", subcore_axis_name="s",
                                  num_cores=2, num_subcores=NSUB)


def _sc_layer_body(hw_hbm, eaw_hbm, b_hbm, src_hbm, dst_hbm, agg_hbm,
                   idx_sa, idx_da, idx_sb, idx_db, idx_st, idx_dt,
                   rows_a, rows_b, eaw_a, eaw_b2, bias_b, acc,
                   sem1, sem2, sem3, sem4):
    c = lax.axis_index("c")
    s = lax.axis_index("s")
    zero = jnp.zeros((LANES,), jnp.float32)

    pltpu.sync_copy(b_hbm, bias_b)
    bias = [bias_b[0, pl.ds(k * LANES, LANES)] for k in range(VPR)]

    def zbody(r, carry):
        for k in range(VPR):
            rows_a[r, pl.ds(k * LANES, LANES)] = zero
        return carry

    lax.fori_loop(0, CHL, zbody, 0)
    base_r = s * RPT

    @pl.when(s < NSUB - 1)
    def _():
        for j in range(6):
            pltpu.sync_copy(rows_a, acc.at[pl.ds(base_r + j * CHL, CHL)])
        pltpu.sync_copy(rows_a.at[pl.ds(0, 48)],
                        acc.at[pl.ds(base_r + 6 * CHL, 48)])

    @pl.when(s == NSUB - 1)
    def _():
        for j in range(6):
            pltpu.sync_copy(rows_a, acc.at[pl.ds(base_r + j * CHL, CHL)])
        pltpu.sync_copy(rows_a.at[pl.ds(0, 64)],
                        acc.at[pl.ds(base_r + 6 * CHL, 64)])

    plsc.subcore_barrier()

    def relu_add(rows, eaw, nrows):
        def body(r, carry):
            for k in range(VPR):
                sl = pl.ds(k * LANES, LANES)
                rows[r, sl] = jnp.maximum((rows[r, sl] + eaw[r, sl])
                                          + bias[k], zero)
            return carry
        lax.fori_loop(0, nrows, body, 0)

    tile_base = c * E + s * EPT

    def drain_scatters():
        pltpu.make_async_copy(hw_hbm.at[pl.ds(0, CHL)], rows_a, sem4).wait()
        pltpu.make_async_copy(hw_hbm.at[pl.ds(0, CHL)], rows_b, sem4).wait()

    def pair(t, carry):
        a = tile_base + (2 * t) * CHL
        b = a + CHL

        @pl.when(t > 0)
        def _():
            drain_scatters()

        cp_sa = pltpu.async_copy(src_hbm.at[pl.ds(a, CHL)], idx_sa, sem1)
        cp_da = pltpu.async_copy(dst_hbm.at[pl.ds(a, CHL)], idx_da, sem2)
        cp_ea = pltpu.async_copy(eaw_hbm.at[pl.ds(a, CHL)], eaw_a, sem3)
        cp_sb = pltpu.async_copy(src_hbm.at[pl.ds(b, CHL)], idx_sb, sem1)
        cp_db = pltpu.async_copy(dst_hbm.at[pl.ds(b, CHL)], idx_db, sem2)
        cp_eb = pltpu.async_copy(eaw_hbm.at[pl.ds(b, CHL)], eaw_b2, sem3)
        cp_sa.wait()
        g_a = pltpu.async_copy(hw_hbm.at[idx_sa], rows_a, sem1)
        cp_sb.wait()
        g_b = pltpu.async_copy(hw_hbm.at[idx_sb], rows_b, sem2)
        g_a.wait()
        cp_ea.wait()
        relu_add(rows_a, eaw_a, CHL)
        cp_da.wait()
        pltpu.async_copy(rows_a, acc.at[idx_da], sem4, add=True)
        g_b.wait()
        cp_eb.wait()
        relu_add(rows_b, eaw_b2, CHL)
        cp_db.wait()
        pltpu.async_copy(rows_b, acc.at[idx_db], sem4, add=True)
        return carry

    lax.fori_loop(0, NPAIR, pair, 0)
    drain_scatters()

    base = tile_base + NPAIR * 2 * CHL
    cp_s = pltpu.async_copy(src_hbm.at[pl.ds(base, TAIL)], idx_st, sem1)
    cp_d = pltpu.async_copy(dst_hbm.at[pl.ds(base, TAIL)], idx_dt, sem2)
    cp_e = pltpu.async_copy(eaw_hbm.at[pl.ds(base, TAIL)],
                            eaw_a.at[pl.ds(0, TAIL)], sem3)
    cp_s.wait()
    pltpu.async_copy(hw_hbm.at[idx_st], rows_a.at[pl.ds(0, TAIL)],
                     sem1).wait()
    cp_e.wait()
    relu_add(rows_a, eaw_a, TAIL)
    cp_d.wait()
    pltpu.sync_copy(rows_a.at[pl.ds(0, TAIL)], acc.at[idx_dt], add=True)

    plsc.subcore_barrier()
    out_base = c * N + base_r

    @pl.when(s < NSUB - 1)
    def _():
        pltpu.sync_copy(acc.at[pl.ds(base_r, RPT)],
                        agg_hbm.at[pl.ds(out_base, RPT)])

    @pl.when(s == NSUB - 1)
    def _():
        pltpu.sync_copy(acc.at[pl.ds(base_r, 640)],
                        agg_hbm.at[pl.ds(out_base, 640)])


@functools.cache
def _sc_layer():
    return pl.kernel(
        _sc_layer_body,
        out_type=[jax.ShapeDtypeStruct((NG, D), jnp.float32)],
        mesh=_mesh(),
        scratch_types=[
            pltpu.VMEM((CHL,), jnp.int32),
            pltpu.VMEM((CHL,), jnp.int32),
            pltpu.VMEM((CHL,), jnp.int32),
            pltpu.VMEM((CHL,), jnp.int32),
            pltpu.VMEM((TAIL,), jnp.int32),
            pltpu.VMEM((TAIL,), jnp.int32),
            pltpu.VMEM((CHL, D), jnp.float32),
            pltpu.VMEM((CHL, D), jnp.float32),
            pltpu.VMEM((CHL, D), jnp.float32),
            pltpu.VMEM((CHL, D), jnp.float32),
            pltpu.VMEM((1, D), jnp.float32),
            pltpu.VMEM_SHARED((N, D), jnp.float32),
            pltpu.SemaphoreType.DMA,
            pltpu.SemaphoreType.DMA,
            pltpu.SemaphoreType.DMA,
            pltpu.SemaphoreType.DMA,
        ],
    )


def _sc_mgather_body(h1_hbm, h2_hbm, h3_hbm, src_hbm, dst_hbm, m_hbm,
                     idx_sa, idx_da, idx_sb, idx_db, idx_st, idx_dt,
                     g1a, g2a, g1b, g2b, sem1, sem2, sem3):
    c = lax.axis_index("c")
    s = lax.axis_index("s")
    wid = c * NSUB + s
    tile_base = wid * EPT_I
    tabs = (h1_hbm, h2_hbm, h3_hbm)

    def add_rows(g1, g2, nrows):
        def rbody(r, carry):
            for t in range(3):
                for k in range(VPR):
                    sl = pl.ds(k * LANES, LANES)
                    g2[t, r, sl] = g1[t, r, sl] + g2[t, r, sl]
            return carry
        lax.fori_loop(0, nrows, rbody, 0)

    def drain_writes():
        for t in range(3):
            pltpu.make_async_copy(m_hbm.at[t].at[pl.ds(0, CHM)],
                                  g2a.at[t], sem3).wait()
            pltpu.make_async_copy(m_hbm.at[t].at[pl.ds(0, CHM)],
                                  g2b.at[t], sem3).wait()

    def pair(i, carry):
        a = tile_base + (2 * i) * CHM
        b = a + CHM

        @pl.when(i > 0)
        def _():
            drain_writes()

        cp_sa = pltpu.async_copy(src_hbm.at[pl.ds(a, CHM)], idx_sa, sem1)
        cp_da = pltpu.async_copy(dst_hbm.at[pl.ds(a, CHM)], idx_da, sem1)
        cp_sb = pltpu.async_copy(src_hbm.at[pl.ds(b, CHM)], idx_sb, sem2)
        cp_db = pltpu.async_copy(dst_hbm.at[pl.ds(b, CHM)], idx_db, sem2)
        cp_sa.wait()
        cp_da.wait()
        ga = ([pltpu.async_copy(tab.at[idx_sa], g1a.at[t], sem1)
               for t, tab in enumerate(tabs)]
              + [pltpu.async_copy(tab.at[idx_da], g2a.at[t], sem1)
                 for t, tab in enumerate(tabs)])
        cp_sb.wait()
        cp_db.wait()
        gb = ([pltpu.async_copy(tab.at[idx_sb], g1b.at[t], sem2)
               for t, tab in enumerate(tabs)]
              + [pltpu.async_copy(tab.at[idx_db], g2b.at[t], sem2)
                 for t, tab in enumerate(tabs)])
        for cp in ga:
            cp.wait()
        add_rows(g1a, g2a, CHM)
        for t in range(3):
            pltpu.async_copy(g2a.at[t], m_hbm.at[t].at[pl.ds(a, CHM)], sem3)
        for cp in gb:
            cp.wait()
        add_rows(g1b, g2b, CHM)
        for t in range(3):
            pltpu.async_copy(g2b.at[t], m_hbm.at[t].at[pl.ds(b, CHM)], sem3)
        return carry

    lax.fori_loop(0, NPAIR_I, pair, 0)
    drain_writes()

    base = tile_base + NPAIR_I * 2 * CHM
    cp_s = pltpu.async_copy(src_hbm.at[pl.ds(base, TAIL_I)], idx_st, sem1)
    cp_d = pltpu.async_copy(dst_hbm.at[pl.ds(base, TAIL_I)], idx_dt, sem2)
    cp_s.wait()
    cp_d.wait()
    gt = ([pltpu.async_copy(tab.at[idx_st], g1a.at[t].at[pl.ds(0, TAIL_I)],
                            sem1) for t, tab in enumerate(tabs)]
          + [pltpu.async_copy(tab.at[idx_dt], g2a.at[t].at[pl.ds(0, TAIL_I)],
                              sem2) for t, tab in enumerate(tabs)])
    for cp in gt:
        cp.wait()
    add_rows(g1a, g2a, TAIL_I)
    for t in range(3):
        pltpu.sync_copy(g2a.at[t].at[pl.ds(0, TAIL_I)],
                        m_hbm.at[t].at[pl.ds(base, TAIL_I)])


@functools.cache
def _sc_mgather():
    return pl.kernel(
        _sc_mgather_body,
        out_type=[jax.ShapeDtypeStruct((3, E, D), jnp.float32)],
        mesh=_mesh(),
        scratch_types=[
            pltpu.VMEM((CHM,), jnp.int32),
            pltpu.VMEM((CHM,), jnp.int32),
            pltpu.VMEM((CHM,), jnp.int32),
            pltpu.VMEM((CHM,), jnp.int32),
            pltpu.VMEM((TAIL_I,), jnp.int32),
            pltpu.VMEM((TAIL_I,), jnp.int32),
            pltpu.VMEM((3, CHM, D), jnp.float32),
            pltpu.VMEM((3, CHM, D), jnp.float32),
            pltpu.VMEM((3, CHM, D), jnp.float32),
            pltpu.VMEM((3, CHM, D), jnp.float32),
            pltpu.SemaphoreType.DMA,
            pltpu.SemaphoreType.DMA,
            pltpu.SemaphoreType.DMA,
        ],
    )


_NODE_BLK = 2000
_EDGE_BLK = 4000
_INTER_BLK = 2000


def _init_body(x_ref, wi_ref, bi_ref, wm_ref, h_ref, hw_ref):
    h = jnp.maximum(
        jnp.dot(x_ref[...], wi_ref[...], preferred_element_type=jnp.float32)
        + bi_ref[...], 0.0)
    h_ref[...] = h
    hw_ref[...] = jnp.dot(h, wm_ref[...], preferred_element_type=jnp.float32)


def _tc_init(x, w_init, b_init, w_msg0):
    nblk = NG // _NODE_BLK
    return pl.pallas_call(
        _init_body,
        grid=(nblk,),
        in_specs=[
            pl.BlockSpec((_NODE_BLK, D), lambda i: (i, 0)),
            pl.BlockSpec((D, D), lambda i: (0, 0)),
            pl.BlockSpec((1, D), lambda i: (0, 0)),
            pl.BlockSpec((D, D), lambda i: (0, 0)),
        ],
        out_specs=[
            pl.BlockSpec((_NODE_BLK, D), lambda i: (i, 0)),
            pl.BlockSpec((_NODE_BLK, D), lambda i: (i, 0)),
        ],
        out_shape=[
            jax.ShapeDtypeStruct((NG, D), jnp.float32),
            jax.ShapeDtypeStruct((NG, D), jnp.float32),
        ],
    )(x, w_init, b_init.reshape(1, D), w_msg0)


def _edge_proj_body(ea_ref, w_ref, o0_ref, o1_ref, o2_ref):
    ea = ea_ref[...]
    w = w_ref[...]
    o0_ref[...] = jnp.dot(ea, w[0:16, :], preferred_element_type=jnp.float32)
    o1_ref[...] = jnp.dot(ea, w[16:32, :], preferred_element_type=jnp.float32)
    o2_ref[...] = jnp.dot(ea, w[32:48, :], preferred_element_type=jnp.float32)


def _tc_edge_proj3(ea, w3):
    n = ea.shape[0]
    nblk = n // _EDGE_BLK
    oblk = lambda: pl.BlockSpec((_EDGE_BLK, D), lambda i: (i, 0))
    return pl.pallas_call(
        _edge_proj_body,
        grid=(nblk,),
        in_specs=[
            pl.BlockSpec((_EDGE_BLK, 16), lambda i: (i, 0)),
            pl.BlockSpec((48, D), lambda i: (0, 0)),
        ],
        out_specs=[oblk(), oblk(), oblk()],
        out_shape=[jax.ShapeDtypeStruct((n, D), jnp.float32)
                   for _ in range(3)],
    )(ea, w3)


def _layer_body(has_msg, *refs):
    refs = list(refs)
    h_ref = refs.pop(0)
    agg_ref = refs.pop(0)
    ws_ref = refs.pop(0)
    wm_ref = refs.pop(0) if has_msg else None
    ho_ref = refs.pop(0)
    hwo_ref = refs.pop(0) if has_msg else None

    hn = jnp.maximum(
        jnp.dot(h_ref[...], ws_ref[...], preferred_element_type=jnp.float32)
        + agg_ref[...], 0.0)
    ho_ref[...] = hn
    if has_msg:
        hwo_ref[...] = jnp.dot(hn, wm_ref[...],
                               preferred_element_type=jnp.float32)


def _tc_layer(h, agg, w_self, w_msg_next):
    has_msg = w_msg_next is not None
    nblk = NG // _NODE_BLK
    blk = lambda: pl.BlockSpec((_NODE_BLK, D), lambda i: (i, 0))
    wblk = lambda: pl.BlockSpec((D, D), lambda i: (0, 0))
    ins = [h, agg, w_self] + ([w_msg_next] if has_msg else [])
    in_specs = [blk(), blk(), wblk()] + ([wblk()] if has_msg else [])
    nouts = 2 if has_msg else 1
    outs = pl.pallas_call(
        functools.partial(_layer_body, has_msg),
        grid=(nblk,),
        in_specs=in_specs,
        out_specs=[blk() for _ in range(nouts)],
        out_shape=[jax.ShapeDtypeStruct((NG, D), jnp.float32)
                   for _ in range(nouts)],
    )(*ins)
    if has_msg:
        return outs[0], outs[1]
    return outs[0], None


def _inter_body(m1_ref, m2_ref, m3_ref, ea_ref, w1_ref, w2_ref, b_ref,
                s_ref, x_ref):
    i = pl.program_id(0)
    w1 = w1_ref[...]
    mw = (jnp.dot(m1_ref[...], w1[0:D, :],
                  preferred_element_type=jnp.float32)
          + jnp.dot(m2_ref[...], w1[D:2 * D, :],
                    preferred_element_type=jnp.float32)
          + jnp.dot(m3_ref[...], w1[2 * D:3 * D, :],
                    preferred_element_type=jnp.float32))
    e = jnp.maximum(
        mw
        + jnp.dot(ea_ref[...], w2_ref[...], preferred_element_type=jnp.float32)
        + b_ref[...], 0.0)
    sm = jnp.sum(e, axis=0, keepdims=True)
    mx = jnp.max(e, axis=0, keepdims=True)

    @pl.when(i == 0)
    def _():
        s_ref[...] = jnp.zeros_like(s_ref)
        x_ref[...] = jnp.zeros_like(x_ref)

    s_ref[...] += jnp.broadcast_to(sm, (8, D))
    x_ref[...] = jnp.maximum(x_ref[...], jnp.broadcast_to(mx, (8, D)))


def _tc_inter(m1, m2, m3, ea, w1, w2, b):
    eblk = lambda: pl.BlockSpec((_INTER_BLK, D), lambda i: (i, 0))
    return pl.pallas_call(
        _inter_body,
        grid=(E // _INTER_BLK,),
        in_specs=[
            eblk(), eblk(), eblk(),
            pl.BlockSpec((_INTER_BLK, 16), lambda i: (i, 0)),
            pl.BlockSpec((3 * D, D), lambda i: (0, 0)),
            pl.BlockSpec((16, D), lambda i: (0, 0)),
            pl.BlockSpec((1, D), lambda i: (0, 0)),
        ],
        out_specs=[pl.BlockSpec((8, D), lambda i: (0, 0)),
                   pl.BlockSpec((8, D), lambda i: (0, 0))],
        out_shape=[jax.ShapeDtypeStruct((8, D), jnp.float32),
                   jax.ShapeDtypeStruct((8, D), jnp.float32)],
    )(m1, m2, m3, ea, w1, w2, b.reshape(1, D))


def _head_body(sp_ref, mp_ref, w1_ref, b1_ref, w2_ref, b2_ref, g_ref, a_ref):
    sm = jnp.sum(sp_ref[...], axis=0, keepdims=True) * (1.0 / (8.0 * E))
    mx = jnp.max(mp_ref[...], axis=0, keepdims=True)
    g = jnp.concatenate([sm, mx], axis=1)
    hfc = jnp.maximum(
        jnp.dot(g, w1_ref[...], preferred_element_type=jnp.float32)
        + b1_ref[...], 0.0)
    aff = jnp.sum(hfc * w2_ref[...], axis=1, keepdims=True) + b2_ref[0, 0]
    g_ref[...] = jnp.broadcast_to(g, (8, 2 * D))
    a_ref[...] = jnp.broadcast_to(aff, (8, D))


def _tc_head(sum_p, max_p, w_fc1, b_fc1, w_fc2, b_fc2):
    full = lambda shape: pl.BlockSpec(shape, lambda: (0, 0))
    return pl.pallas_call(
        _head_body,
        in_specs=[
            full((8, D)), full((8, D)),
            full((2 * D, 2 * D)), full((1, 2 * D)),
            full((1, 2 * D)), full((1, 1)),
        ],
        out_specs=[full((8, 2 * D)), full((8, D))],
        out_shape=[jax.ShapeDtypeStruct((8, 2 * D), jnp.float32),
                   jax.ShapeDtypeStruct((8, D), jnp.float32)],
    )(sum_p, max_p, w_fc1, b_fc1.reshape(1, 2 * D),
      w_fc2.reshape(1, 2 * D), b_fc2.reshape(1, 1))


def kernel(x_lig, edge_index_lig, edge_attr_lig, x_prot, edge_index_prot,
           edge_attr_prot, edge_index_inter, edge_attr_inter, W_init, b_init,
           W_msg_h, W_msg_e, b_msg, W_self, W_inter_m, W_inter_e, b_inter,
           W_fc1, b_fc1, W_fc2, b_fc2):
    x_all = jnp.concatenate([x_lig, x_prot], axis=0)
    src_all = jnp.concatenate(
        [edge_index_lig[0], edge_index_prot[0] + N]).astype(jnp.int32)
    dst_all = jnp.concatenate(
        [edge_index_lig[1], edge_index_prot[1]]).astype(jnp.int32)
    ea_all = jnp.concatenate([edge_attr_lig, edge_attr_prot], axis=0)
    src_i = edge_index_inter[0].astype(jnp.int32)
    dst_i = edge_index_inter[1].astype(jnp.int32)

    h, hw = _tc_init(x_all, W_init, b_init, W_msg_h[0])
    eaw = _tc_edge_proj3(ea_all, W_msg_e.reshape(48, D))

    hs = []
    for l in range(3):
        (agg,) = _sc_layer()(hw, eaw[l], b_msg[l].reshape(1, D),
                             src_all, dst_all)
        w_msg_next = W_msg_h[l + 1] if l < 2 else None
        h, hw = _tc_layer(h, agg, W_self[l], w_msg_next)
        hs.append(h)

    (m,) = _sc_mgather()(hs[0], hs[1], hs[2], src_i, dst_i)
    sum_p, max_p = _tc_inter(m[0], m[1], m[2], edge_attr_inter, W_inter_m,
                             W_inter_e, b_inter)

    g8, a8 = _tc_head(sum_p, max_p, W_fc1, b_fc1, W_fc2, b_fc2)
    affinity_pred = a8[0:1, 0:1]
    g = g8[0:1, :]
    ranking = jnp.zeros((1,), jnp.float32)
    return (affinity_pred, g, ranking)


# speedup vs baseline: 2.7410x; 1.0201x over previous
import functools

import jax
import jax.numpy as jnp
from jax import lax
from jax.experimental import pallas as pl
from jax.experimental.pallas import tpu as pltpu
from jax.experimental.pallas import tpu_sc as plsc

N = 10000
NG = 2 * N
E = 160000
D = 128
LANES = 16
NSUB = 16
VPR = D // LANES

CH = 128
EPT = E // NSUB
CHL = 96
NPAIR = EPT // (2 * CHL)
TAIL = EPT - NPAIR * 2 * CHL
RPT = 624

EPT_I = E // (2 * NSUB)
CHM = 64
NPAIR_I = EPT_I // (2 * CHM)
TAIL_I = EPT_I - NPAIR_I * 2 * CHM


@functools.cache
def _mesh():
    return plsc.VectorSubcoreMesh(core_axis_name="c", subcore_axis_name="s",
                                  num_cores=2, num_subcores=NSUB)


def _sc_layer_body(hw_hbm, eaw_hbm, b_hbm, src_hbm, dst_hbm, agg_hbm,
                   idx_sa, idx_da, idx_sb, idx_db, idx_st, idx_dt,
                   rows_a, rows_b, eaw_a, eaw_b2, bias_b, acc,
                   sem1, sem2, sem3, sem4):
    c = lax.axis_index("c")
    s = lax.axis_index("s")
    zero = jnp.zeros((LANES,), jnp.float32)

    pltpu.sync_copy(b_hbm, bias_b)
    bias = [bias_b[0, pl.ds(k * LANES, LANES)] for k in range(VPR)]

    def zbody(r, carry):
        for k in range(VPR):
            rows_a[r, pl.ds(k * LANES, LANES)] = zero
        return carry

    lax.fori_loop(0, CHL, zbody, 0)
    base_r = s * RPT

    @pl.when(s < NSUB - 1)
    def _():
        for j in range(6):
            pltpu.sync_copy(rows_a, acc.at[pl.ds(base_r + j * CHL, CHL)])
        pltpu.sync_copy(rows_a.at[pl.ds(0, 48)],
                        acc.at[pl.ds(base_r + 6 * CHL, 48)])

    @pl.when(s == NSUB - 1)
    def _():
        for j in range(6):
            pltpu.sync_copy(rows_a, acc.at[pl.ds(base_r + j * CHL, CHL)])
        pltpu.sync_copy(rows_a.at[pl.ds(0, 64)],
                        acc.at[pl.ds(base_r + 6 * CHL, 64)])

    plsc.subcore_barrier()

    def relu_add(rows, eaw, nrows):
        def body(r, carry):
            for k in range(VPR):
                sl = pl.ds(k * LANES, LANES)
                rows[r, sl] = jnp.maximum((rows[r, sl] + eaw[r, sl])
                                          + bias[k], zero)
            return carry
        lax.fori_loop(0, nrows, body, 0)

    tile_base = c * E + s * EPT

    def drain_scatters():
        pltpu.make_async_copy(hw_hbm.at[pl.ds(0, CHL)], rows_a, sem4).wait()
        pltpu.make_async_copy(hw_hbm.at[pl.ds(0, CHL)], rows_b, sem4).wait()

    def pair(t, carry):
        a = tile_base + (2 * t) * CHL
        b = a + CHL

        @pl.when(t > 0)
        def _():
            drain_scatters()

        cp_sa = pltpu.async_copy(src_hbm.at[pl.ds(a, CHL)], idx_sa, sem1)
        cp_da = pltpu.async_copy(dst_hbm.at[pl.ds(a, CHL)], idx_da, sem2)
        cp_ea = pltpu.async_copy(eaw_hbm.at[pl.ds(a, CHL)], eaw_a, sem3)
        cp_sb = pltpu.async_copy(src_hbm.at[pl.ds(b, CHL)], idx_sb, sem1)
        cp_db = pltpu.async_copy(dst_hbm.at[pl.ds(b, CHL)], idx_db, sem2)
        cp_eb = pltpu.async_copy(eaw_hbm.at[pl.ds(b, CHL)], eaw_b2, sem3)
        cp_sa.wait()
        g_a = pltpu.async_copy(hw_hbm.at[idx_sa], rows_a, sem1)
        cp_sb.wait()
        g_b = pltpu.async_copy(hw_hbm.at[idx_sb], rows_b, sem2)
        g_a.wait()
        cp_ea.wait()
        relu_add(rows_a, eaw_a, CHL)
        cp_da.wait()
        pltpu.async_copy(rows_a, acc.at[idx_da], sem4, add=True)
        g_b.wait()
        cp_eb.wait()
        relu_add(rows_b, eaw_b2, CHL)
        cp_db.wait()
        pltpu.async_copy(rows_b, acc.at[idx_db], sem4, add=True)
        return carry

    lax.fori_loop(0, NPAIR, pair, 0)
    drain_scatters()

    base = tile_base + NPAIR * 2 * CHL
    cp_s = pltpu.async_copy(src_hbm.at[pl.ds(base, TAIL)], idx_st, sem1)
    cp_d = pltpu.async_copy(dst_hbm.at[pl.ds(base, TAIL)], idx_dt, sem2)
    cp_e = pltpu.async_copy(eaw_hbm.at[pl.ds(base, TAIL)],
                            eaw_a.at[pl.ds(0, TAIL)], sem3)
    cp_s.wait()
    pltpu.async_copy(hw_hbm.at[idx_st], rows_a.at[pl.ds(0, TAIL)],
                     sem1).wait()
    cp_e.wait()
    relu_add(rows_a, eaw_a, TAIL)
    cp_d.wait()
    pltpu.sync_copy(rows_a.at[pl.ds(0, TAIL)], acc.at[idx_dt], add=True)

    plsc.subcore_barrier()
    out_base = c * N + base_r

    @pl.when(s < NSUB - 1)
    def _():
        pltpu.sync_copy(acc.at[pl.ds(base_r, RPT)],
                        agg_hbm.at[pl.ds(out_base, RPT)])

    @pl.when(s == NSUB - 1)
    def _():
        pltpu.sync_copy(acc.at[pl.ds(base_r, 640)],
                        agg_hbm.at[pl.ds(out_base, 640)])


@functools.cache
def _sc_layer():
    return pl.kernel(
        _sc_layer_body,
        out_type=[jax.ShapeDtypeStruct((NG, D), jnp.float32)],
        mesh=_mesh(),
        scratch_types=[
            pltpu.VMEM((CHL,), jnp.int32),
            pltpu.VMEM((CHL,), jnp.int32),
            pltpu.VMEM((CHL,), jnp.int32),
            pltpu.VMEM((CHL,), jnp.int32),
            pltpu.VMEM((TAIL,), jnp.int32),
            pltpu.VMEM((TAIL,), jnp.int32),
            pltpu.VMEM((CHL, D), jnp.float32),
            pltpu.VMEM((CHL, D), jnp.float32),
            pltpu.VMEM((CHL, D), jnp.float32),
            pltpu.VMEM((CHL, D), jnp.float32),
            pltpu.VMEM((1, D), jnp.float32),
            pltpu.VMEM_SHARED((N, D), jnp.float32),
            pltpu.SemaphoreType.DMA,
            pltpu.SemaphoreType.DMA,
            pltpu.SemaphoreType.DMA,
            pltpu.SemaphoreType.DMA,
        ],
    )


def _sc_mgather_body(h1_hbm, h2_hbm, h3_hbm, src_hbm, dst_hbm, m_hbm,
                     idx_sa, idx_da, idx_sb, idx_db, idx_st, idx_dt,
                     g1a, g2a, g1b, g2b, sem1, sem2, sem3):
    c = lax.axis_index("c")
    s = lax.axis_index("s")
    wid = c * NSUB + s
    tile_base = wid * EPT_I
    tabs = (h1_hbm, h2_hbm, h3_hbm)

    def add_rows(g1, g2, nrows):
        def rbody(r, carry):
            for t in range(3):
                for k in range(VPR):
                    sl = pl.ds(k * LANES, LANES)
                    g2[t, r, sl] = g1[t, r, sl] + g2[t, r, sl]
            return carry
        lax.fori_loop(0, nrows, rbody, 0)

    def drain_writes():
        for t in range(3):
            pltpu.make_async_copy(m_hbm.at[t].at[pl.ds(0, CHM)],
                                  g2a.at[t], sem3).wait()
            pltpu.make_async_copy(m_hbm.at[t].at[pl.ds(0, CHM)],
                                  g2b.at[t], sem3).wait()

    def pair(i, carry):
        a = tile_base + (2 * i) * CHM
        b = a + CHM

        @pl.when(i > 0)
        def _():
            drain_writes()

        cp_sa = pltpu.async_copy(src_hbm.at[pl.ds(a, CHM)], idx_sa, sem1)
        cp_da = pltpu.async_copy(dst_hbm.at[pl.ds(a, CHM)], idx_da, sem1)
        cp_sb = pltpu.async_copy(src_hbm.at[pl.ds(b, CHM)], idx_sb, sem2)
        cp_db = pltpu.async_copy(dst_hbm.at[pl.ds(b, CHM)], idx_db, sem2)
        cp_sa.wait()
        cp_da.wait()
        ga = ([pltpu.async_copy(tab.at[idx_sa], g1a.at[t], sem1)
               for t, tab in enumerate(tabs)]
              + [pltpu.async_copy(tab.at[idx_da], g2a.at[t], sem1)
                 for t, tab in enumerate(tabs)])
        cp_sb.wait()
        cp_db.wait()
        gb = ([pltpu.async_copy(tab.at[idx_sb], g1b.at[t], sem2)
               for t, tab in enumerate(tabs)]
              + [pltpu.async_copy(tab.at[idx_db], g2b.at[t], sem2)
                 for t, tab in enumerate(tabs)])
        for cp in ga:
            cp.wait()
        add_rows(g1a, g2a, CHM)
        for t in range(3):
            pltpu.async_copy(g2a.at[t], m_hbm.at[t].at[pl.ds(a, CHM)], sem3)
        for cp in gb:
            cp.wait()
        add_rows(g1b, g2b, CHM)
        for t in range(3):
            pltpu.async_copy(g2b.at[t], m_hbm.at[t].at[pl.ds(b, CHM)], sem3)
        return carry

    lax.fori_loop(0, NPAIR_I, pair, 0)
    drain_writes()

    base = tile_base + NPAIR_I * 2 * CHM
    cp_s = pltpu.async_copy(src_hbm.at[pl.ds(base, TAIL_I)], idx_st, sem1)
    cp_d = pltpu.async_copy(dst_hbm.at[pl.ds(base, TAIL_I)], idx_dt, sem2)
    cp_s.wait()
    cp_d.wait()
    gt = ([pltpu.async_copy(tab.at[idx_st], g1a.at[t].at[pl.ds(0, TAIL_I)],
                            sem1) for t, tab in enumerate(tabs)]
          + [pltpu.async_copy(tab.at[idx_dt], g2a.at[t].at[pl.ds(0, TAIL_I)],
                              sem2) for t, tab in enumerate(tabs)])
    for cp in gt:
        cp.wait()
    add_rows(g1a, g2a, TAIL_I)
    for t in range(3):
        pltpu.sync_copy(g2a.at[t].at[pl.ds(0, TAIL_I)],
                        m_hbm.at[t].at[pl.ds(base, TAIL_I)])


@functools.cache
def _sc_mgather():
    return pl.kernel(
        _sc_mgather_body,
        out_type=[jax.ShapeDtypeStruct((3, E, D), jnp.float32)],
        mesh=_mesh(),
        scratch_types=[
            pltpu.VMEM((CHM,), jnp.int32),
            pltpu.VMEM((CHM,), jnp.int32),
            pltpu.VMEM((CHM,), jnp.int32),
            pltpu.VMEM((CHM,), jnp.int32),
            pltpu.VMEM((TAIL_I,), jnp.int32),
            pltpu.VMEM((TAIL_I,), jnp.int32),
            pltpu.VMEM((3, CHM, D), jnp.float32),
            pltpu.VMEM((3, CHM, D), jnp.float32),
            pltpu.VMEM((3, CHM, D), jnp.float32),
            pltpu.VMEM((3, CHM, D), jnp.float32),
            pltpu.SemaphoreType.DMA,
            pltpu.SemaphoreType.DMA,
            pltpu.SemaphoreType.DMA,
        ],
    )


_NODE_BLK = 2000
_EDGE_BLK = 4000
_INTER_BLK = 2000


def _init_body(x_ref, wi_ref, bi_ref, wm_ref, h_ref, hw_ref):
    h = jnp.maximum(
        jnp.dot(x_ref[...], wi_ref[...], preferred_element_type=jnp.float32)
        + bi_ref[...], 0.0)
    h_ref[...] = h
    hw_ref[...] = jnp.dot(h, wm_ref[...], preferred_element_type=jnp.float32)


def _tc_init(x, w_init, b_init, w_msg0):
    nblk = NG // _NODE_BLK
    return pl.pallas_call(
        _init_body,
        grid=(nblk,),
        in_specs=[
            pl.BlockSpec((_NODE_BLK, D), lambda i: (i, 0)),
            pl.BlockSpec((D, D), lambda i: (0, 0)),
            pl.BlockSpec((1, D), lambda i: (0, 0)),
            pl.BlockSpec((D, D), lambda i: (0, 0)),
        ],
        out_specs=[
            pl.BlockSpec((_NODE_BLK, D), lambda i: (i, 0)),
            pl.BlockSpec((_NODE_BLK, D), lambda i: (i, 0)),
        ],
        out_shape=[
            jax.ShapeDtypeStruct((NG, D), jnp.float32),
            jax.ShapeDtypeStruct((NG, D), jnp.float32),
        ],
    )(x, w_init, b_init.reshape(1, D), w_msg0)


def _edge_proj_body(ea_ref, w_ref, o_ref):
    o_ref[...] = jnp.dot(ea_ref[...], w_ref[...],
                         preferred_element_type=jnp.float32)


def _tc_edge_proj(ea, w):
    n = ea.shape[0]
    nblk = n // _EDGE_BLK
    return pl.pallas_call(
        _edge_proj_body,
        grid=(nblk,),
        in_specs=[
            pl.BlockSpec((_EDGE_BLK, 16), lambda i: (i, 0)),
            pl.BlockSpec((16, D), lambda i: (0, 0)),
        ],
        out_specs=pl.BlockSpec((_EDGE_BLK, D), lambda i: (i, 0)),
        out_shape=jax.ShapeDtypeStruct((n, D), jnp.float32),
    )(ea, w)


def _layer_body(has_msg, *refs):
    refs = list(refs)
    h_ref = refs.pop(0)
    agg_ref = refs.pop(0)
    ws_ref = refs.pop(0)
    wm_ref = refs.pop(0) if has_msg else None
    ho_ref = refs.pop(0)
    hwo_ref = refs.pop(0) if has_msg else None

    hn = jnp.maximum(
        jnp.dot(h_ref[...], ws_ref[...], preferred_element_type=jnp.float32)
        + agg_ref[...], 0.0)
    ho_ref[...] = hn
    if has_msg:
        hwo_ref[...] = jnp.dot(hn, wm_ref[...],
                               preferred_element_type=jnp.float32)


def _tc_layer(h, agg, w_self, w_msg_next):
    has_msg = w_msg_next is not None
    nblk = NG // _NODE_BLK
    blk = lambda: pl.BlockSpec((_NODE_BLK, D), lambda i: (i, 0))
    wblk = lambda: pl.BlockSpec((D, D), lambda i: (0, 0))
    ins = [h, agg, w_self] + ([w_msg_next] if has_msg else [])
    in_specs = [blk(), blk(), wblk()] + ([wblk()] if has_msg else [])
    nouts = 2 if has_msg else 1
    outs = pl.pallas_call(
        functools.partial(_layer_body, has_msg),
        grid=(nblk,),
        in_specs=in_specs,
        out_specs=[blk() for _ in range(nouts)],
        out_shape=[jax.ShapeDtypeStruct((NG, D), jnp.float32)
                   for _ in range(nouts)],
    )(*ins)
    if has_msg:
        return outs[0], outs[1]
    return outs[0], None


def _inter_body(m1_ref, m2_ref, m3_ref, ea_ref, w1_ref, w2_ref, b_ref,
                s_ref, x_ref):
    i = pl.program_id(0)
    w1 = w1_ref[...]
    mw = (jnp.dot(m1_ref[...], w1[0:D, :],
                  preferred_element_type=jnp.float32)
          + jnp.dot(m2_ref[...], w1[D:2 * D, :],
                    preferred_element_type=jnp.float32)
          + jnp.dot(m3_ref[...], w1[2 * D:3 * D, :],
                    preferred_element_type=jnp.float32))
    e = jnp.maximum(
        mw
        + jnp.dot(ea_ref[...], w2_ref[...], preferred_element_type=jnp.float32)
        + b_ref[...], 0.0)
    sm = jnp.sum(e, axis=0, keepdims=True)
    mx = jnp.max(e, axis=0, keepdims=True)

    @pl.when(i == 0)
    def _():
        s_ref[...] = jnp.zeros_like(s_ref)
        x_ref[...] = jnp.zeros_like(x_ref)

    s_ref[...] += jnp.broadcast_to(sm, (8, D))
    x_ref[...] = jnp.maximum(x_ref[...], jnp.broadcast_to(mx, (8, D)))


def _tc_inter(m1, m2, m3, ea, w1, w2, b):
    eblk = lambda: pl.BlockSpec((_INTER_BLK, D), lambda i: (i, 0))
    return pl.pallas_call(
        _inter_body,
        grid=(E // _INTER_BLK,),
        in_specs=[
            eblk(), eblk(), eblk(),
            pl.BlockSpec((_INTER_BLK, 16), lambda i: (i, 0)),
            pl.BlockSpec((3 * D, D), lambda i: (0, 0)),
            pl.BlockSpec((16, D), lambda i: (0, 0)),
            pl.BlockSpec((1, D), lambda i: (0, 0)),
        ],
        out_specs=[pl.BlockSpec((8, D), lambda i: (0, 0)),
                   pl.BlockSpec((8, D), lambda i: (0, 0))],
        out_shape=[jax.ShapeDtypeStruct((8, D), jnp.float32),
                   jax.ShapeDtypeStruct((8, D), jnp.float32)],
    )(m1, m2, m3, ea, w1, w2, b.reshape(1, D))


def _head_body(sp_ref, mp_ref, w1_ref, b1_ref, w2_ref, b2_ref, g_ref, a_ref):
    sm = jnp.sum(sp_ref[...], axis=0, keepdims=True) * (1.0 / (8.0 * E))
    mx = jnp.max(mp_ref[...], axis=0, keepdims=True)
    g = jnp.concatenate([sm, mx], axis=1)
    hfc = jnp.maximum(
        jnp.dot(g, w1_ref[...], preferred_element_type=jnp.float32)
        + b1_ref[...], 0.0)
    aff = jnp.sum(hfc * w2_ref[...], axis=1, keepdims=True) + b2_ref[0, 0]
    g_ref[...] = jnp.broadcast_to(g, (8, 2 * D))
    a_ref[...] = jnp.broadcast_to(aff, (8, D))


def _tc_head(sum_p, max_p, w_fc1, b_fc1, w_fc2, b_fc2):
    full = lambda shape: pl.BlockSpec(shape, lambda: (0, 0))
    return pl.pallas_call(
        _head_body,
        in_specs=[
            full((8, D)), full((8, D)),
            full((2 * D, 2 * D)), full((1, 2 * D)),
            full((1, 2 * D)), full((1, 1)),
        ],
        out_specs=[full((8, 2 * D)), full((8, D))],
        out_shape=[jax.ShapeDtypeStruct((8, 2 * D), jnp.float32),
                   jax.ShapeDtypeStruct((8, D), jnp.float32)],
    )(sum_p, max_p, w_fc1, b_fc1.reshape(1, 2 * D),
      w_fc2.reshape(1, 2 * D), b_fc2.reshape(1, 1))


def kernel(x_lig, edge_index_lig, edge_attr_lig, x_prot, edge_index_prot,
           edge_attr_prot, edge_index_inter, edge_attr_inter, W_init, b_init,
           W_msg_h, W_msg_e, b_msg, W_self, W_inter_m, W_inter_e, b_inter,
           W_fc1, b_fc1, W_fc2, b_fc2):
    x_all = jnp.concatenate([x_lig, x_prot], axis=0)
    src_all = jnp.concatenate(
        [edge_index_lig[0], edge_index_prot[0] + N]).astype(jnp.int32)
    dst_all = jnp.concatenate(
        [edge_index_lig[1], edge_index_prot[1]]).astype(jnp.int32)
    ea_all = jnp.concatenate([edge_attr_lig, edge_attr_prot], axis=0)
    src_i = edge_index_inter[0].astype(jnp.int32)
    dst_i = edge_index_inter[1].astype(jnp.int32)

    h, hw = _tc_init(x_all, W_init, b_init, W_msg_h[0])

    hs = []
    for l in range(3):
        eaw_l = _tc_edge_proj(ea_all, W_msg_e[l])
        (agg,) = _sc_layer()(hw, eaw_l, b_msg[l].reshape(1, D),
                             src_all, dst_all)
        w_msg_next = W_msg_h[l + 1] if l < 2 else None
        h, hw = _tc_layer(h, agg, W_self[l], w_msg_next)
        hs.append(h)

    (m,) = _sc_mgather()(hs[0], hs[1], hs[2], src_i, dst_i)
    sum_p, max_p = _tc_inter(m[0], m[1], m[2], edge_attr_inter, W_inter_m,
                             W_inter_e, b_inter)

    g8, a8 = _tc_head(sum_p, max_p, W_fc1, b_fc1, W_fc2, b_fc2)
    affinity_pred = a8[0:1, 0:1]
    g = g8[0:1, :]
    ranking = jnp.zeros((1,), jnp.float32)
    return (affinity_pred, g, ranking)
